# Initial kernel scaffold; baseline (speedup 1.0000x reference)
import jax
import jax.numpy as jnp
from jax.experimental import pallas as pl


def kernel(x, x_vector_attr, edge_attr, edge_vector_attr, sse_attr, sse_vector_attr, params, edge_index, node_to_sse, batch):
    raise NotImplementedError("write your pallas kernel here")


# speedup vs baseline: 8.8823x; 8.8823x over previous
import functools

import jax
import jax.numpy as jnp
from jax import lax
from jax.experimental import pallas as pl
from jax.experimental.pallas import tpu as pltpu
from jax.experimental.pallas import tpu_sc as plsc

N = 50000
E = 800000
NSSE = 5000
NG = 50

NW = 32
NCORE = 2
NSUB = 16
CH = 128

NPAD = 53248
EPAD = 819200
SPAD = 5120
GPAD = 128

F32 = jnp.float32


def _ln_math(s, V, g, b):
    mu = jnp.mean(s, axis=-1, keepdims=True)
    var = jnp.mean((s - mu) ** 2, axis=-1, keepdims=True)
    s = (s - mu) * lax.rsqrt(var + 1e-5) * g + b
    vn2 = V[0] * V[0] + V[1] * V[1] + V[2] * V[2]
    vnorm = jnp.sqrt(jnp.mean(vn2, axis=-1, keepdims=True) + 1e-8)
    V = [v / vnorm for v in V]
    return s, V


def _gvp_math(s, V, Wh, Ws, bs, Wv=None, Wg=None, bg=None, act=None):
    Vh = [jnp.dot(v, Wh, preferred_element_type=F32) for v in V]
    vn = jnp.sqrt(Vh[0] ** 2 + Vh[1] ** 2 + Vh[2] ** 2 + 1e-8)
    so = jnp.dot(jnp.concatenate([s, vn], axis=-1), Ws,
                 preferred_element_type=F32) + bs
    Vout = None
    if Wv is not None:
        Vout = [jnp.dot(vh, Wv, preferred_element_type=F32) for vh in Vh]
        gin = act(so) if act is not None else so
        gate = jax.nn.sigmoid(jnp.dot(gin, Wg, preferred_element_type=F32) + bg)
        Vout = [v * gate for v in Vout]
    if act is not None:
        so = act(so)
    return so, Vout


def _pack128(so, Vo):
    b = so.shape[0]
    return jnp.concatenate(
        [so] + Vo + [jnp.ones((b, 1), F32), jnp.zeros((b, 39), F32)], axis=-1)


def _group6(packed):
    return jnp.stack([packed[:, 16 * g:16 * (g + 1)] for g in range(6)],
                     axis=0)


def _rspec(blk, f):
    return pl.BlockSpec((blk, f), lambda i: (i, 0))


def _fspec(shape):
    nd = len(shape)
    return pl.BlockSpec(shape, lambda i: (0,) * nd)


_TC_PARAMS = pltpu.CompilerParams(dimension_semantics=("parallel",))


def _tc_init(s_in, v_in, lg, lb, Wh, Ws, bs, Wv, Wg, bg, nrows, blk, si, vi):
    def body(s_ref, v_ref, lg_r, lb_r, wh_r, ws_r, bs_r, wv_r, wg_r, bg_r,
             o_ref):
        s = s_ref[...]
        V = [v_ref[:, j * vi:(j + 1) * vi] for j in range(3)]
        s, V = _ln_math(s, V, lg_r[...], lb_r[...])
        so, Vo = _gvp_math(s, V, wh_r[...], ws_r[...], bs_r[...],
                           wv_r[...], wg_r[...], bg_r[...], act=None)
        o_ref[...] = _pack128(so, Vo)

    ws = [lg, lb, Wh, Ws, bs, Wv, Wg, bg]
    return pl.pallas_call(
        body, grid=(nrows // blk,),
        in_specs=[_rspec(blk, si), _rspec(blk, 3 * vi)] +
                 [_fspec(w.shape) for w in ws],
        out_specs=_rspec(blk, 128),
        out_shape=jax.ShapeDtypeStruct((nrows, 128), F32),
        compiler_params=_TC_PARAMS,
    )(s_in, v_in, *ws)


def _tc_edge_init(ea, ev8, lg, lb, Wh, Ws, bs, Wv, Wg, bg, blk=1024):
    def body(a_ref, v_ref, lg_r, lb_r, wh_r, ws_r, bs_r, wv_r, wg_r, bg_r,
             so_ref, vo_ref):
        s = a_ref[...]
        V = [v_ref[:, j:j + 1] for j in range(3)]
        s, V = _ln_math(s, V, lg_r[...], lb_r[...])
        so, Vo = _gvp_math(s, V, wh_r[...], ws_r[...], bs_r[...],
                           wv_r[...], wg_r[...], bg_r[...], act=None)
        so_ref[...] = so
        vo_ref[...] = jnp.concatenate(
            Vo + [jnp.zeros((so.shape[0], 5), F32)], axis=-1)

    ws = [lg, lb, Wh, Ws, bs, Wv, Wg, bg]
    return pl.pallas_call(
        body, grid=(EPAD // blk,),
        in_specs=[_rspec(blk, 32), _rspec(blk, 8)] +
                 [_fspec(w.shape) for w in ws],
        out_specs=[_rspec(blk, 32), _rspec(blk, 8)],
        out_shape=[jax.ShapeDtypeStruct((EPAD, 32), F32),
                   jax.ShapeDtypeStruct((EPAD, 8), F32)],
        compiler_params=_TC_PARAMS,
    )(ea, ev8, *ws)


def _tc_msg(gs, gd, es, ev8, Wh, Ws, bs, Wv, Wg, bg, blk=1024):
    def body(gs_ref, gd_ref, es_ref, ev_ref, wh_r, ws_r, bs_r, wv_r, wg_r,
             bg_r, o_ref):
        a = gs_ref[...]
        b = gd_ref[...]
        s = jnp.concatenate([a[:, :64], es_ref[...], b[:, :64]], axis=-1)
        ev = ev_ref[...]
        V = [jnp.concatenate([a[:, 64 + 8 * j:72 + 8 * j], ev[:, j:j + 1],
                              b[:, 64 + 8 * j:72 + 8 * j]], axis=-1)
             for j in range(3)]
        so, Vo = _gvp_math(s, V, wh_r[...], ws_r[...], bs_r[...],
                           wv_r[...], wg_r[...], bg_r[...], act=jax.nn.relu)
        nb = so.shape[0]
        packed = jnp.concatenate(
            [so] + Vo + [jnp.ones((nb, 1), F32), jnp.zeros((nb, 7), F32)],
            axis=-1)
        o_ref[...] = _group6(packed)

    ws = [Wh, Ws, bs, Wv, Wg, bg]
    return pl.pallas_call(
        body, grid=(EPAD // blk,),
        in_specs=[_rspec(blk, 128), _rspec(blk, 128), _rspec(blk, 32),
                  _rspec(blk, 8)] + [_fspec(w.shape) for w in ws],
        out_specs=pl.BlockSpec((6, blk, 16), lambda i: (0, i, 0)),
        out_shape=jax.ShapeDtypeStruct((6, EPAD, 16), F32),
        compiler_params=_TC_PARAMS,
    )(gs, gd, es, ev8, *ws)


def _mean_from_partials(m_ref):
    m = m_ref[...]
    m = m[0] + m[1]
    cnt = jnp.maximum(m[5][:, 8:9], 1.0)
    ms = jnp.concatenate([m[0], m[1], m[2], m[3]], axis=-1) / cnt
    mV = [m[4][:, 0:8] / cnt, m[4][:, 8:16] / cnt, m[5][:, 0:8] / cnt]
    return ms, mV


def _tc_node_upd(H, Msum, l1g, l1b, Wh, Ws, bs, Wv, Wg, bg, l2g, l2b,
                 blk=1024):
    def body(h_ref, m_ref, l1g_r, l1b_r, wh_r, ws_r, bs_r, wv_r, wg_r, bg_r,
             l2g_r, l2b_r, o_ref, og_ref):
        h = h_ref[...]
        ms, mV = _mean_from_partials(m_ref)
        s = h[:, :64] + ms
        V = [h[:, 64 + 8 * j:72 + 8 * j] + mV[j] for j in range(3)]
        s, V = _ln_math(s, V, l1g_r[...], l1b_r[...])
        ds, dV = _gvp_math(s, V, wh_r[...], ws_r[...], bs_r[...],
                           wv_r[...], wg_r[...], bg_r[...], act=jax.nn.relu)
        s2, V2 = _ln_math(s + ds, [V[j] + dV[j] for j in range(3)],
                          l2g_r[...], l2b_r[...])
        packed = _pack128(s2, V2)
        o_ref[...] = packed
        og_ref[...] = _group6(packed)

    ws = [l1g, l1b, Wh, Ws, bs, Wv, Wg, bg, l2g, l2b]
    return pl.pallas_call(
        body, grid=(NPAD // blk,),
        in_specs=[_rspec(blk, 128),
                  pl.BlockSpec((2, 6, blk, 16), lambda i: (0, 0, i, 0))] +
                 [_fspec(w.shape) for w in ws],
        out_specs=[_rspec(blk, 128),
                   pl.BlockSpec((6, blk, 16), lambda i: (0, i, 0))],
        out_shape=[jax.ShapeDtypeStruct((NPAD, 128), F32),
                   jax.ShapeDtypeStruct((6, NPAD, 16), F32)],
        compiler_params=_TC_PARAMS,
    )(H, Msum, *ws)


def _tc_sse_upd(SS, Pool, Wh, Ws, bs, Wv, Wg, bg, lg, lb, blk=640):
    def body(ss_ref, p_ref, wh_r, ws_r, bs_r, wv_r, wg_r, bg_r, lg_r, lb_r,
             o_ref):
        h = ss_ref[...]
        ps, pV = _mean_from_partials(p_ref)
        s = jnp.concatenate([h[:, :64], ps], axis=-1)
        V = [jnp.concatenate([h[:, 64 + 8 * j:72 + 8 * j], pV[j]], axis=-1)
             for j in range(3)]
        ds, dV = _gvp_math(s, V, wh_r[...], ws_r[...], bs_r[...],
                           wv_r[...], wg_r[...], bg_r[...], act=jax.nn.relu)
        s2, V2 = _ln_math(h[:, :64] + ds,
                          [h[:, 64 + 8 * j:72 + 8 * j] + dV[j]
                           for j in range(3)], lg_r[...], lb_r[...])
        o_ref[...] = _pack128(s2, V2)

    ws = [Wh, Ws, bs, Wv, Wg, bg, lg, lb]
    return pl.pallas_call(
        body, grid=(SPAD // blk,),
        in_specs=[_rspec(blk, 128),
                  pl.BlockSpec((2, 6, blk, 16), lambda i: (0, 0, i, 0))] +
                 [_fspec(w.shape) for w in ws],
        out_specs=_rspec(blk, 128),
        out_shape=jax.ShapeDtypeStruct((SPAD, 128), F32),
        compiler_params=_TC_PARAMS,
    )(SS, Pool, *ws)


def _tc_node_sse(H2, Bc, Wh, Ws, bs, Wv, Wg, bg, lg, lb, blk=1024):
    def body(h_ref, b_ref, wh_r, ws_r, bs_r, wv_r, wg_r, bg_r, lg_r, lb_r,
             o_ref):
        h = h_ref[...]
        c = b_ref[...]
        s = jnp.concatenate([h[:, :64], c[:, :64]], axis=-1)
        V = [jnp.concatenate([h[:, 64 + 8 * j:72 + 8 * j],
                              c[:, 64 + 8 * j:72 + 8 * j]], axis=-1)
             for j in range(3)]
        ds, dV = _gvp_math(s, V, wh_r[...], ws_r[...], bs_r[...],
                           wv_r[...], wg_r[...], bg_r[...], act=jax.nn.relu)
        s2, V2 = _ln_math(h[:, :64] + ds,
                          [h[:, 64 + 8 * j:72 + 8 * j] + dV[j]
                           for j in range(3)], lg_r[...], lb_r[...])
        o_ref[...] = _pack128(s2, V2)

    ws = [Wh, Ws, bs, Wv, Wg, bg, lg, lb]
    return pl.pallas_call(
        body, grid=(NPAD // blk,),
        in_specs=[_rspec(blk, 128), _rspec(blk, 128)] +
                 [_fspec(w.shape) for w in ws],
        out_specs=_rspec(blk, 128),
        out_shape=jax.ShapeDtypeStruct((NPAD, 128), F32),
        compiler_params=_TC_PARAMS,
    )(H2, Bc, *ws)


def _tc_out(H, lg, lb, Wh, Ws, bs, blk=1024):
    def body(h_ref, lg_r, lb_r, wh_r, ws_r, bs_r, o_ref, og_ref):
        h = h_ref[...]
        s = h[:, :64]
        V = [h[:, 64 + 8 * j:72 + 8 * j] for j in range(3)]
        s, V = _ln_math(s, V, lg_r[...], lb_r[...])
        so, _ = _gvp_math(s, V, wh_r[...], ws_r[...], bs_r[...],
                          act=jax.nn.relu)
        o_ref[...] = so
        og_ref[...] = jnp.stack([so[:, 16 * g:16 * (g + 1)] for g in range(4)], axis=0)

    ws = [lg, lb, Wh, Ws, bs]
    return pl.pallas_call(
        body, grid=(NPAD // blk,),
        in_specs=[_rspec(blk, 128)] + [_fspec(w.shape) for w in ws],
        out_specs=[_rspec(blk, 64),
                   pl.BlockSpec((4, blk, 16), lambda i: (0, i, 0))],
        out_shape=[jax.ShapeDtypeStruct((NPAD, 64), F32),
                   jax.ShapeDtypeStruct((4, NPAD, 16), F32)],
        compiler_params=_TC_PARAMS,
    )(H, *ws)


def _tc_graph_combine(Gp):
    def body(g_ref, o_ref):
        g = g_ref[...]
        g = g[0] + g[1]
        o_ref[...] = jnp.concatenate([g[0], g[1], g[2], g[3]], axis=-1)

    return pl.pallas_call(
        body, grid=(1,),
        in_specs=[_fspec((2, 4, GPAD, 16))],
        out_specs=_fspec((GPAD, 64)),
        out_shape=jax.ShapeDtypeStruct((GPAD, 64), F32),
    )(Gp)


_SC_MESH = dict(core_axis_name="c", subcore_axis_name="s")


def _sc_gather(table, idx, nrows_out):
    per_w = nrows_out // NW
    nch = per_w // CH
    fdim = table.shape[1]

    @functools.partial(
        pl.kernel,
        mesh=plsc.VectorSubcoreMesh(**_SC_MESH),
        out_type=jax.ShapeDtypeStruct((nrows_out, fdim), F32),
        scratch_types=[pltpu.VMEM((CH,), jnp.int32),
                       pltpu.VMEM((CH, fdim), F32),
                       pltpu.SemaphoreType.DMA],
    )
    def k(t_ref, i_ref, o_ref, idx_v, rows_v, sem):
        wid = lax.axis_index("s") * NCORE + lax.axis_index("c")
        base = wid * per_w

        def body(j, carry):
            off = base + j * CH
            pltpu.sync_copy(i_ref.at[pl.ds(off, CH)], idx_v)
            pltpu.async_copy(t_ref.at[idx_v], rows_v, sem).wait()
            pltpu.sync_copy(rows_v, o_ref.at[pl.ds(off, CH)])
            return carry

        lax.fori_loop(0, nch, body, 0)

    return k(table, idx)


def _sc_scatter(msgs, idx, table_rows, nsrc, ngroups):
    per_w = nsrc // NW
    nch = per_w // CH
    rpt = table_rows // NSUB
    zeros = jnp.zeros((rpt, 16), F32)

    @functools.partial(
        pl.kernel,
        mesh=plsc.VectorSubcoreMesh(**_SC_MESH),
        out_type=jax.ShapeDtypeStruct((NCORE, ngroups, table_rows, 16), F32),
        scratch_types=[pltpu.VMEM((CH,), jnp.int32),
                       pltpu.VMEM((CH, 16), F32),
                       pltpu.VMEM_SHARED((table_rows, 16), F32)],
        compiler_params=pltpu.CompilerParams(use_tc_tiling_on_sc=False),
    )
    def k(m_ref, i_ref, z_ref, o_ref, idx_c, m_v, shared):
        cid = lax.axis_index("c")
        sid = lax.axis_index("s")
        wid = sid * NCORE + cid
        for g in range(ngroups):
            pltpu.sync_copy(z_ref, shared.at[pl.ds(sid * rpt, rpt)])
            plsc.subcore_barrier()

            def body(j, carry):
                off = wid * per_w + j * CH
                pltpu.sync_copy(i_ref.at[pl.ds(off, CH)], idx_c)
                pltpu.sync_copy(m_ref.at[g, pl.ds(off, CH)], m_v)
                pltpu.sync_copy(m_v, shared.at[idx_c], add=True)
                return carry

            lax.fori_loop(0, nch, body, 0)
            plsc.subcore_barrier()
            pltpu.sync_copy(shared.at[pl.ds(sid * rpt, rpt)],
                            o_ref.at[cid, g, pl.ds(sid * rpt, rpt)])
            plsc.subcore_barrier()

    return k(msgs, idx, zeros)


def kernel(x, x_vector_attr, edge_attr, edge_vector_attr, sse_attr,
           sse_vector_attr, params, edge_index, node_to_sse, batch):
    xs = jnp.pad(x, ((0, NPAD - N), (0, 0)))
    xv3 = jnp.pad(x_vector_attr.transpose(0, 2, 1).reshape(N, 12),
                  ((0, NPAD - N), (0, 0)))
    eas = jnp.pad(edge_attr, ((0, EPAD - E), (0, 0)))
    ev8 = jnp.pad(edge_vector_attr.reshape(E, 3), ((0, EPAD - E), (0, 5)))
    sss = jnp.pad(sse_attr, ((0, SPAD - NSSE), (0, 0)))
    ssv3 = jnp.pad(sse_vector_attr.transpose(0, 2, 1).reshape(NSSE, 12),
                   ((0, SPAD - NSSE), (0, 0)))

    src = jnp.pad(edge_index[0].astype(jnp.int32), (0, EPAD - E))
    dstg = jnp.pad(edge_index[1].astype(jnp.int32), (0, EPAD - E))
    dst_sc = jnp.pad(edge_index[1].astype(jnp.int32), (0, EPAD - E),
                     constant_values=N)
    n2s = node_to_sse.astype(jnp.int32)
    n2s_g = jnp.pad(n2s, (0, NPAD - N))
    n2s_sc = jnp.pad(n2s, (0, NPAD - N), constant_values=NSSE)
    bat_sc = jnp.pad(batch.astype(jnp.int32), (0, NPAD - N),
                     constant_values=NG)

    r2 = lambda w: w.reshape(1, -1)

    def LN(p):
        return r2(p['g']), r2(p['b'])

    def GW(p):
        return (p['Wh'], p['Ws'], r2(p['bs']), p['Wv'], p['Wg'], r2(p['bg']))

    H = _tc_init(xs, xv3, *LN(params['W_v']['ln']),
                 *GW(params['W_v']['gvp']), NPAD, 1024, 64, 4)
    es_a, ev_a = _tc_edge_init(eas, ev8, *LN(params['W_e']['ln']),
                               *GW(params['W_e']['gvp']))
    SS = _tc_init(sss, ssv3, *LN(params['W_sse']['ln']),
                  *GW(params['W_sse']['gvp']), SPAD, 640, 64, 4)

    for lp in params['layers']:
        Gs = _sc_gather(H, src, EPAD)
        Gd = _sc_gather(H, dstg, EPAD)
        M = _tc_msg(Gs, Gd, es_a, ev_a, *GW(lp['msg']))
        Msum = _sc_scatter(M, dst_sc, NPAD, EPAD, 6)
        H2, H2g = _tc_node_upd(H, Msum, *LN(lp['ln1']), *GW(lp['ff']),
                               *LN(lp['ln2']))
        Pool = _sc_scatter(H2g, n2s_sc, SPAD, NPAD, 6)
        SS = _tc_sse_upd(SS, Pool, *GW(lp['sse_upd']), *LN(lp['ln_sse']))
        Bc = _sc_gather(SS, n2s_g, NPAD)
        H = _tc_node_sse(H2, Bc, *GW(lp['node_sse']), *LN(lp['ln3']))

    ne, neg = _tc_out(H, *LN(params['W_out']['ln']),
                      params['W_out']['gvp']['Wh'],
                      params['W_out']['gvp']['Ws'],
                      r2(params['W_out']['gvp']['bs']))
    Gp = _sc_scatter(neg, bat_sc, GPAD, NPAD, 4)
    ge = _tc_graph_combine(Gp)
    return ne[:N], ge[:NG]


# speedup vs baseline: 13.7600x; 1.5491x over previous
import functools

import jax
import jax.numpy as jnp
from jax import lax
from jax.experimental import pallas as pl
from jax.experimental.pallas import tpu as pltpu
from jax.experimental.pallas import tpu_sc as plsc

N = 50000
E = 800000
NSSE = 5000
NG = 50

NW = 32
NCORE = 2
NSUB = 16
CH = 128

NPAD = 53248
EPAD = 819200
SPAD = 5120
GPAD = 128

F32 = jnp.float32


def _ln_math(s, V, g, b):
    mu = jnp.mean(s, axis=-1, keepdims=True)
    var = jnp.mean((s - mu) ** 2, axis=-1, keepdims=True)
    s = (s - mu) * lax.rsqrt(var + 1e-5) * g + b
    vn2 = V[0] * V[0] + V[1] * V[1] + V[2] * V[2]
    vnorm = jnp.sqrt(jnp.mean(vn2, axis=-1, keepdims=True) + 1e-8)
    V = [v / vnorm for v in V]
    return s, V


def _gvp_math(s, V, Wh, Ws, bs, Wv=None, Wg=None, bg=None, act=None):
    Vh = [jnp.dot(v, Wh, preferred_element_type=F32) for v in V]
    vn = jnp.sqrt(Vh[0] ** 2 + Vh[1] ** 2 + Vh[2] ** 2 + 1e-8)
    so = jnp.dot(jnp.concatenate([s, vn], axis=-1), Ws,
                 preferred_element_type=F32) + bs
    Vout = None
    if Wv is not None:
        Vout = [jnp.dot(vh, Wv, preferred_element_type=F32) for vh in Vh]
        gin = act(so) if act is not None else so
        gate = jax.nn.sigmoid(jnp.dot(gin, Wg, preferred_element_type=F32) + bg)
        Vout = [v * gate for v in Vout]
    if act is not None:
        so = act(so)
    return so, Vout


def _pack128(so, Vo):
    b = so.shape[0]
    return jnp.concatenate(
        [so] + Vo + [jnp.ones((b, 1), F32), jnp.zeros((b, 39), F32)], axis=-1)


def _rspec(blk, f):
    return pl.BlockSpec((blk, f), lambda i: (i, 0))


def _fspec(shape):
    nd = len(shape)
    return pl.BlockSpec(shape, lambda i: (0,) * nd)


_TC_PARAMS = pltpu.CompilerParams(dimension_semantics=("parallel",))


def _tc_init(s_in, v_in, lg, lb, Wh, Ws, bs, Wv, Wg, bg, nrows, blk, si, vi):
    def body(s_ref, v_ref, lg_r, lb_r, wh_r, ws_r, bs_r, wv_r, wg_r, bg_r,
             o_ref):
        s = s_ref[...]
        V = [v_ref[:, j * vi:(j + 1) * vi] for j in range(3)]
        s, V = _ln_math(s, V, lg_r[...], lb_r[...])
        so, Vo = _gvp_math(s, V, wh_r[...], ws_r[...], bs_r[...],
                           wv_r[...], wg_r[...], bg_r[...], act=None)
        o_ref[...] = _pack128(so, Vo)

    ws = [lg, lb, Wh, Ws, bs, Wv, Wg, bg]
    return pl.pallas_call(
        body, grid=(nrows // blk,),
        in_specs=[_rspec(blk, si), _rspec(blk, 3 * vi)] +
                 [_fspec(w.shape) for w in ws],
        out_specs=_rspec(blk, 128),
        out_shape=jax.ShapeDtypeStruct((nrows, 128), F32),
        compiler_params=_TC_PARAMS,
    )(s_in, v_in, *ws)


def _tc_edge_init(ea, ev8, lg, lb, Wh, Ws, bs, Wv, Wg, bg, blk=1024):
    def body(a_ref, v_ref, lg_r, lb_r, wh_r, ws_r, bs_r, wv_r, wg_r, bg_r,
             so_ref, vo_ref):
        s = a_ref[...]
        V = [v_ref[:, j:j + 1] for j in range(3)]
        s, V = _ln_math(s, V, lg_r[...], lb_r[...])
        so, Vo = _gvp_math(s, V, wh_r[...], ws_r[...], bs_r[...],
                           wv_r[...], wg_r[...], bg_r[...], act=None)
        so_ref[...] = so
        vo_ref[...] = jnp.concatenate(
            Vo + [jnp.zeros((so.shape[0], 5), F32)], axis=-1)

    ws = [lg, lb, Wh, Ws, bs, Wv, Wg, bg]
    return pl.pallas_call(
        body, grid=(EPAD // blk,),
        in_specs=[_rspec(blk, 32), _rspec(blk, 8)] +
                 [_fspec(w.shape) for w in ws],
        out_specs=[_rspec(blk, 32), _rspec(blk, 8)],
        out_shape=[jax.ShapeDtypeStruct((EPAD, 32), F32),
                   jax.ShapeDtypeStruct((EPAD, 8), F32)],
        compiler_params=_TC_PARAMS,
    )(ea, ev8, *ws)


def _tc_msg(gs, gd, es, ev8, Wh, Ws, bs, Wv, Wg, bg, blk=1024):
    def body(gs_ref, gd_ref, es_ref, ev_ref, wh_r, ws_r, bs_r, wv_r, wg_r,
             bg_r, o_ref):
        a = gs_ref[...]
        b = gd_ref[...]
        s = jnp.concatenate([a[:, :64], es_ref[...], b[:, :64]], axis=-1)
        ev = ev_ref[...]
        V = [jnp.concatenate([a[:, 64 + 8 * j:72 + 8 * j], ev[:, j:j + 1],
                              b[:, 64 + 8 * j:72 + 8 * j]], axis=-1)
             for j in range(3)]
        so, Vo = _gvp_math(s, V, wh_r[...], ws_r[...], bs_r[...],
                           wv_r[...], wg_r[...], bg_r[...], act=jax.nn.relu)
        o_ref[...] = _pack128(so, Vo)

    ws = [Wh, Ws, bs, Wv, Wg, bg]
    return pl.pallas_call(
        body, grid=(EPAD // blk,),
        in_specs=[_rspec(blk, 128), _rspec(blk, 128), _rspec(blk, 32),
                  _rspec(blk, 8)] + [_fspec(w.shape) for w in ws],
        out_specs=_rspec(blk, 128),
        out_shape=jax.ShapeDtypeStruct((EPAD, 128), F32),
        compiler_params=_TC_PARAMS,
    )(gs, gd, es, ev8, *ws)


def _mean_from_partials(m_ref):
    m = m_ref[...]
    m = m[0] + m[1]
    cnt = jnp.maximum(m[5][:, 8:9], 1.0)
    ms = jnp.concatenate([m[0], m[1], m[2], m[3]], axis=-1) / cnt
    mV = [m[4][:, 0:8] / cnt, m[4][:, 8:16] / cnt, m[5][:, 0:8] / cnt]
    return ms, mV


def _tc_node_upd(H, Msum, l1g, l1b, Wh, Ws, bs, Wv, Wg, bg, l2g, l2b,
                 blk=1024):
    def body(h_ref, m_ref, l1g_r, l1b_r, wh_r, ws_r, bs_r, wv_r, wg_r, bg_r,
             l2g_r, l2b_r, o_ref):
        h = h_ref[...]
        ms, mV = _mean_from_partials(m_ref)
        s = h[:, :64] + ms
        V = [h[:, 64 + 8 * j:72 + 8 * j] + mV[j] for j in range(3)]
        s, V = _ln_math(s, V, l1g_r[...], l1b_r[...])
        ds, dV = _gvp_math(s, V, wh_r[...], ws_r[...], bs_r[...],
                           wv_r[...], wg_r[...], bg_r[...], act=jax.nn.relu)
        s2, V2 = _ln_math(s + ds, [V[j] + dV[j] for j in range(3)],
                          l2g_r[...], l2b_r[...])
        o_ref[...] = _pack128(s2, V2)

    ws = [l1g, l1b, Wh, Ws, bs, Wv, Wg, bg, l2g, l2b]
    return pl.pallas_call(
        body, grid=(NPAD // blk,),
        in_specs=[_rspec(blk, 128),
                  pl.BlockSpec((2, 6, blk, 16), lambda i: (0, 0, i, 0))] +
                 [_fspec(w.shape) for w in ws],
        out_specs=_rspec(blk, 128),
        out_shape=jax.ShapeDtypeStruct((NPAD, 128), F32),
        compiler_params=_TC_PARAMS,
    )(H, Msum, *ws)


def _tc_sse_upd(SS, Pool, Wh, Ws, bs, Wv, Wg, bg, lg, lb, blk=640):
    def body(ss_ref, p_ref, wh_r, ws_r, bs_r, wv_r, wg_r, bg_r, lg_r, lb_r,
             o_ref):
        h = ss_ref[...]
        ps, pV = _mean_from_partials(p_ref)
        s = jnp.concatenate([h[:, :64], ps], axis=-1)
        V = [jnp.concatenate([h[:, 64 + 8 * j:72 + 8 * j], pV[j]], axis=-1)
             for j in range(3)]
        ds, dV = _gvp_math(s, V, wh_r[...], ws_r[...], bs_r[...],
                           wv_r[...], wg_r[...], bg_r[...], act=jax.nn.relu)
        s2, V2 = _ln_math(h[:, :64] + ds,
                          [h[:, 64 + 8 * j:72 + 8 * j] + dV[j]
                           for j in range(3)], lg_r[...], lb_r[...])
        o_ref[...] = _pack128(s2, V2)

    ws = [Wh, Ws, bs, Wv, Wg, bg, lg, lb]
    return pl.pallas_call(
        body, grid=(SPAD // blk,),
        in_specs=[_rspec(blk, 128),
                  pl.BlockSpec((2, 6, blk, 16), lambda i: (0, 0, i, 0))] +
                 [_fspec(w.shape) for w in ws],
        out_specs=_rspec(blk, 128),
        out_shape=jax.ShapeDtypeStruct((SPAD, 128), F32),
        compiler_params=_TC_PARAMS,
    )(SS, Pool, *ws)


def _tc_node_sse(H2, Bc, Wh, Ws, bs, Wv, Wg, bg, lg, lb, blk=1024):
    def body(h_ref, b_ref, wh_r, ws_r, bs_r, wv_r, wg_r, bg_r, lg_r, lb_r,
             o_ref):
        h = h_ref[...]
        c = b_ref[...]
        s = jnp.concatenate([h[:, :64], c[:, :64]], axis=-1)
        V = [jnp.concatenate([h[:, 64 + 8 * j:72 + 8 * j],
                              c[:, 64 + 8 * j:72 + 8 * j]], axis=-1)
             for j in range(3)]
        ds, dV = _gvp_math(s, V, wh_r[...], ws_r[...], bs_r[...],
                           wv_r[...], wg_r[...], bg_r[...], act=jax.nn.relu)
        s2, V2 = _ln_math(h[:, :64] + ds,
                          [h[:, 64 + 8 * j:72 + 8 * j] + dV[j]
                           for j in range(3)], lg_r[...], lb_r[...])
        o_ref[...] = _pack128(s2, V2)

    ws = [Wh, Ws, bs, Wv, Wg, bg, lg, lb]
    return pl.pallas_call(
        body, grid=(NPAD // blk,),
        in_specs=[_rspec(blk, 128), _rspec(blk, 128)] +
                 [_fspec(w.shape) for w in ws],
        out_specs=_rspec(blk, 128),
        out_shape=jax.ShapeDtypeStruct((NPAD, 128), F32),
        compiler_params=_TC_PARAMS,
    )(H2, Bc, *ws)


def _tc_out(H, lg, lb, Wh, Ws, bs, blk=1024):
    def body(h_ref, lg_r, lb_r, wh_r, ws_r, bs_r, o_ref, og_ref):
        h = h_ref[...]
        s = h[:, :64]
        V = [h[:, 64 + 8 * j:72 + 8 * j] for j in range(3)]
        s, V = _ln_math(s, V, lg_r[...], lb_r[...])
        so, _ = _gvp_math(s, V, wh_r[...], ws_r[...], bs_r[...],
                          act=jax.nn.relu)
        o_ref[...] = so
        og_ref[...] = jnp.concatenate(
            [so, jnp.zeros((so.shape[0], 64), F32)], axis=-1)

    ws = [lg, lb, Wh, Ws, bs]
    return pl.pallas_call(
        body, grid=(NPAD // blk,),
        in_specs=[_rspec(blk, 128)] + [_fspec(w.shape) for w in ws],
        out_specs=[_rspec(blk, 64), _rspec(blk, 128)],
        out_shape=[jax.ShapeDtypeStruct((NPAD, 64), F32),
                   jax.ShapeDtypeStruct((NPAD, 128), F32)],
        compiler_params=_TC_PARAMS,
    )(H, *ws)


def _tc_graph_combine(Gp):
    def body(g_ref, o_ref):
        g = g_ref[...]
        g = g[0] + g[1]
        o_ref[...] = jnp.concatenate([g[0], g[1], g[2], g[3]], axis=-1)

    return pl.pallas_call(
        body, grid=(1,),
        in_specs=[_fspec((2, 4, GPAD, 16))],
        out_specs=_fspec((GPAD, 64)),
        out_shape=jax.ShapeDtypeStruct((GPAD, 64), F32),
    )(Gp)


_SC_MESH = dict(core_axis_name="c", subcore_axis_name="s")


def _sc_gather(table, idx, nrows_out):
    per_w = nrows_out // NW
    nch = per_w // CH
    fdim = table.shape[1]

    @functools.partial(
        pl.kernel,
        mesh=plsc.VectorSubcoreMesh(**_SC_MESH),
        out_type=jax.ShapeDtypeStruct((nrows_out, fdim), F32),
        scratch_types=[pltpu.VMEM((CH,), jnp.int32),
                       pltpu.VMEM((CH, fdim), F32),
                       pltpu.SemaphoreType.DMA],
    )
    def k(t_ref, i_ref, o_ref, idx_v, rows_v, sem):
        wid = lax.axis_index("s") * NCORE + lax.axis_index("c")
        base = wid * per_w

        def body(j, carry):
            off = base + j * CH
            pltpu.sync_copy(i_ref.at[pl.ds(off, CH)], idx_v)
            pltpu.async_copy(t_ref.at[idx_v], rows_v, sem).wait()
            pltpu.sync_copy(rows_v, o_ref.at[pl.ds(off, CH)])
            return carry

        lax.fori_loop(0, nch, body, 0)

    return k(table, idx)


def _sc_scatter(msgs, idx, table_rows, nsrc, ngroups):
    per_w = nsrc // NW
    nch = per_w // CH
    rpt = table_rows // NSUB
    zeros = jnp.zeros((rpt, 16), F32)

    @functools.partial(
        pl.kernel,
        mesh=plsc.VectorSubcoreMesh(**_SC_MESH),
        out_type=jax.ShapeDtypeStruct((NCORE, ngroups, table_rows, 16), F32),
        scratch_types=[pltpu.VMEM((CH,), jnp.int32),
                       pltpu.VMEM((CH, 16), F32),
                       pltpu.VMEM_SHARED((table_rows, 16), F32)],
        compiler_params=pltpu.CompilerParams(use_tc_tiling_on_sc=False),
    )
    def k(m_ref, i_ref, z_ref, o_ref, idx_c, m_v, shared):
        cid = lax.axis_index("c")
        sid = lax.axis_index("s")
        wid = sid * NCORE + cid
        for g in range(ngroups):
            pltpu.sync_copy(z_ref, shared.at[pl.ds(sid * rpt, rpt)])
            plsc.subcore_barrier()

            def body(j, carry):
                off = wid * per_w + j * CH
                pltpu.sync_copy(i_ref.at[pl.ds(off, CH)], idx_c)
                pltpu.sync_copy(
                    m_ref.at[pl.ds(off, CH), pl.ds(16 * g, 16)], m_v)
                pltpu.sync_copy(m_v, shared.at[idx_c], add=True)
                return carry

            lax.fori_loop(0, nch, body, 0)
            plsc.subcore_barrier()
            pltpu.sync_copy(shared.at[pl.ds(sid * rpt, rpt)],
                            o_ref.at[cid, g, pl.ds(sid * rpt, rpt)])
            plsc.subcore_barrier()

    return k(msgs, idx, zeros)


def kernel(x, x_vector_attr, edge_attr, edge_vector_attr, sse_attr,
           sse_vector_attr, params, edge_index, node_to_sse, batch):
    xs = jnp.pad(x, ((0, NPAD - N), (0, 0)))
    xv3 = jnp.pad(x_vector_attr.transpose(0, 2, 1).reshape(N, 12),
                  ((0, NPAD - N), (0, 0)))
    eas = jnp.pad(edge_attr, ((0, EPAD - E), (0, 0)))
    ev8 = jnp.pad(edge_vector_attr.reshape(E, 3), ((0, EPAD - E), (0, 5)))
    sss = jnp.pad(sse_attr, ((0, SPAD - NSSE), (0, 0)))
    ssv3 = jnp.pad(sse_vector_attr.transpose(0, 2, 1).reshape(NSSE, 12),
                   ((0, SPAD - NSSE), (0, 0)))

    epad_i = jnp.arange(EPAD - E, dtype=jnp.int32)
    npad_i = jnp.arange(NPAD - N, dtype=jnp.int32)
    src = jnp.concatenate([edge_index[0].astype(jnp.int32), epad_i % N])
    dstg = jnp.concatenate([edge_index[1].astype(jnp.int32), epad_i % N])
    dst_sc = jnp.concatenate([edge_index[1].astype(jnp.int32),
                              N + epad_i % (NPAD - N)])
    n2s = node_to_sse.astype(jnp.int32)
    n2s_g = jnp.concatenate([n2s, npad_i % NSSE])
    n2s_sc = jnp.concatenate([n2s, NSSE + npad_i % (SPAD - NSSE)])
    bat_sc = jnp.concatenate([batch.astype(jnp.int32),
                              NG + npad_i % (GPAD - NG)])

    r2 = lambda w: w.reshape(1, -1)

    def LN(p):
        return r2(p['g']), r2(p['b'])

    def GW(p):
        return (p['Wh'], p['Ws'], r2(p['bs']), p['Wv'], p['Wg'], r2(p['bg']))

    H = _tc_init(xs, xv3, *LN(params['W_v']['ln']),
                 *GW(params['W_v']['gvp']), NPAD, 1024, 64, 4)
    es_a, ev_a = _tc_edge_init(eas, ev8, *LN(params['W_e']['ln']),
                               *GW(params['W_e']['gvp']))
    SS = _tc_init(sss, ssv3, *LN(params['W_sse']['ln']),
                  *GW(params['W_sse']['gvp']), SPAD, 640, 64, 4)

    for lp in params['layers']:
        Gs = _sc_gather(H, src, EPAD)
        Gd = _sc_gather(H, dstg, EPAD)
        M = _tc_msg(Gs, Gd, es_a, ev_a, *GW(lp['msg']))
        Msum = _sc_scatter(M, dst_sc, NPAD, EPAD, 6)
        H2 = _tc_node_upd(H, Msum, *LN(lp['ln1']), *GW(lp['ff']),
                          *LN(lp['ln2']))
        Pool = _sc_scatter(H2, n2s_sc, SPAD, NPAD, 6)
        SS = _tc_sse_upd(SS, Pool, *GW(lp['sse_upd']), *LN(lp['ln_sse']))
        Bc = _sc_gather(SS, n2s_g, NPAD)
        H = _tc_node_sse(H2, Bc, *GW(lp['node_sse']), *LN(lp['ln3']))

    ne, neg = _tc_out(H, *LN(params['W_out']['ln']),
                      params['W_out']['gvp']['Wh'],
                      params['W_out']['gvp']['Ws'],
                      r2(params['W_out']['gvp']['bs']))
    Gp = _sc_scatter(neg, bat_sc, GPAD, NPAD, 4)
    ge = _tc_graph_combine(Gp)
    return ne[:N], ge[:NG]


# speedup vs baseline: 16.4920x; 1.1985x over previous
import functools

import jax
import jax.numpy as jnp
from jax import lax
from jax.experimental import pallas as pl
from jax.experimental.pallas import tpu as pltpu
from jax.experimental.pallas import tpu_sc as plsc

N = 50000
E = 800000
NSSE = 5000
NG = 50

NW = 32
NCORE = 2
NSUB = 16
CH = 128

NPAD = 53248
EPAD = 819200
SPAD = 5120
GPAD = 128

F32 = jnp.float32


def _ln_math(s, V, g, b):
    mu = jnp.mean(s, axis=-1, keepdims=True)
    var = jnp.mean((s - mu) ** 2, axis=-1, keepdims=True)
    s = (s - mu) * lax.rsqrt(var + 1e-5) * g + b
    vn2 = V[0] * V[0] + V[1] * V[1] + V[2] * V[2]
    vnorm = jnp.sqrt(jnp.mean(vn2, axis=-1, keepdims=True) + 1e-8)
    V = [v / vnorm for v in V]
    return s, V


def _gvp_math(s, V, Wh, Ws, bs, Wv=None, Wg=None, bg=None, act=None):
    Vh = [jnp.dot(v, Wh, preferred_element_type=F32) for v in V]
    vn = jnp.sqrt(Vh[0] ** 2 + Vh[1] ** 2 + Vh[2] ** 2 + 1e-8)
    so = jnp.dot(jnp.concatenate([s, vn], axis=-1), Ws,
                 preferred_element_type=F32) + bs
    Vout = None
    if Wv is not None:
        Vout = [jnp.dot(vh, Wv, preferred_element_type=F32) for vh in Vh]
        gin = act(so) if act is not None else so
        gate = jax.nn.sigmoid(jnp.dot(gin, Wg, preferred_element_type=F32) + bg)
        Vout = [v * gate for v in Vout]
    if act is not None:
        so = act(so)
    return so, Vout


def _pack128(so, Vo):
    b = so.shape[0]
    return jnp.concatenate(
        [so] + Vo + [jnp.ones((b, 1), F32), jnp.zeros((b, 39), F32)], axis=-1)


def _rspec(blk, f):
    return pl.BlockSpec((blk, f), lambda i: (i, 0))


def _fspec(shape):
    nd = len(shape)
    return pl.BlockSpec(shape, lambda i: (0,) * nd)


_TC_PARAMS = pltpu.CompilerParams(dimension_semantics=("parallel",))


def _tc_init(s_in, v_in, lg, lb, Wh, Ws, bs, Wv, Wg, bg, nrows, blk, si, vi):
    def body(s_ref, v_ref, lg_r, lb_r, wh_r, ws_r, bs_r, wv_r, wg_r, bg_r,
             o_ref):
        s = s_ref[...]
        V = [v_ref[:, j * vi:(j + 1) * vi] for j in range(3)]
        s, V = _ln_math(s, V, lg_r[...], lb_r[...])
        so, Vo = _gvp_math(s, V, wh_r[...], ws_r[...], bs_r[...],
                           wv_r[...], wg_r[...], bg_r[...], act=None)
        o_ref[...] = _pack128(so, Vo)

    ws = [lg, lb, Wh, Ws, bs, Wv, Wg, bg]
    return pl.pallas_call(
        body, grid=(nrows // blk,),
        in_specs=[_rspec(blk, si), _rspec(blk, 3 * vi)] +
                 [_fspec(w.shape) for w in ws],
        out_specs=_rspec(blk, 128),
        out_shape=jax.ShapeDtypeStruct((nrows, 128), F32),
        compiler_params=_TC_PARAMS,
    )(s_in, v_in, *ws)


def _tc_edge_init(ea, ev8, lg, lb, Wh, Ws, bs, Wv, Wg, bg, blk=1024):
    def body(a_ref, v_ref, lg_r, lb_r, wh_r, ws_r, bs_r, wv_r, wg_r, bg_r,
             so_ref, vo_ref):
        s = a_ref[...]
        V = [v_ref[:, j:j + 1] for j in range(3)]
        s, V = _ln_math(s, V, lg_r[...], lb_r[...])
        so, Vo = _gvp_math(s, V, wh_r[...], ws_r[...], bs_r[...],
                           wv_r[...], wg_r[...], bg_r[...], act=None)
        so_ref[...] = so
        vo_ref[...] = jnp.concatenate(
            Vo + [jnp.zeros((so.shape[0], 5), F32)], axis=-1)

    ws = [lg, lb, Wh, Ws, bs, Wv, Wg, bg]
    return pl.pallas_call(
        body, grid=(EPAD // blk,),
        in_specs=[_rspec(blk, 32), _rspec(blk, 8)] +
                 [_fspec(w.shape) for w in ws],
        out_specs=[_rspec(blk, 32), _rspec(blk, 8)],
        out_shape=[jax.ShapeDtypeStruct((EPAD, 32), F32),
                   jax.ShapeDtypeStruct((EPAD, 8), F32)],
        compiler_params=_TC_PARAMS,
    )(ea, ev8, *ws)


def _tc_msg(gs, gd, es, ev8, Wh, Ws, bs, Wv, Wg, bg, blk=1024):
    def body(gs_ref, gd_ref, es_ref, ev_ref, wh_r, ws_r, bs_r, wv_r, wg_r,
             bg_r, o_ref):
        a = gs_ref[...]
        b = gd_ref[...]
        s = jnp.concatenate([a[:, :64], es_ref[...], b[:, :64]], axis=-1)
        ev = ev_ref[...]
        V = [jnp.concatenate([a[:, 64 + 8 * j:72 + 8 * j], ev[:, j:j + 1],
                              b[:, 64 + 8 * j:72 + 8 * j]], axis=-1)
             for j in range(3)]
        so, Vo = _gvp_math(s, V, wh_r[...], ws_r[...], bs_r[...],
                           wv_r[...], wg_r[...], bg_r[...], act=jax.nn.relu)
        o_ref[...] = _pack128(so, Vo)

    ws = [Wh, Ws, bs, Wv, Wg, bg]
    return pl.pallas_call(
        body, grid=(EPAD // blk,),
        in_specs=[_rspec(blk, 128), _rspec(blk, 128), _rspec(blk, 32),
                  _rspec(blk, 8)] + [_fspec(w.shape) for w in ws],
        out_specs=_rspec(blk, 128),
        out_shape=jax.ShapeDtypeStruct((EPAD, 128), F32),
        compiler_params=_TC_PARAMS,
    )(gs, gd, es, ev8, *ws)


def _mean_from_partials(m_ref):
    m = m_ref[...]
    m = m[0] + m[1]
    cnt = jnp.maximum(m[:, 88:89], 1.0)
    ms = m[:, :64] / cnt
    mV = [m[:, 64 + 8 * j:72 + 8 * j] / cnt for j in range(3)]
    return ms, mV


def _tc_node_upd(H, Msum, l1g, l1b, Wh, Ws, bs, Wv, Wg, bg, l2g, l2b,
                 blk=1024):
    def body(h_ref, m_ref, l1g_r, l1b_r, wh_r, ws_r, bs_r, wv_r, wg_r, bg_r,
             l2g_r, l2b_r, o_ref):
        h = h_ref[...]
        ms, mV = _mean_from_partials(m_ref)
        s = h[:, :64] + ms
        V = [h[:, 64 + 8 * j:72 + 8 * j] + mV[j] for j in range(3)]
        s, V = _ln_math(s, V, l1g_r[...], l1b_r[...])
        ds, dV = _gvp_math(s, V, wh_r[...], ws_r[...], bs_r[...],
                           wv_r[...], wg_r[...], bg_r[...], act=jax.nn.relu)
        s2, V2 = _ln_math(s + ds, [V[j] + dV[j] for j in range(3)],
                          l2g_r[...], l2b_r[...])
        o_ref[...] = _pack128(s2, V2)

    ws = [l1g, l1b, Wh, Ws, bs, Wv, Wg, bg, l2g, l2b]
    return pl.pallas_call(
        body, grid=(NPAD // blk,),
        in_specs=[_rspec(blk, 128),
                  pl.BlockSpec((2, blk, 128), lambda i: (0, i, 0))] +
                 [_fspec(w.shape) for w in ws],
        out_specs=_rspec(blk, 128),
        out_shape=jax.ShapeDtypeStruct((NPAD, 128), F32),
        compiler_params=_TC_PARAMS,
    )(H, Msum, *ws)


def _tc_sse_upd(SS, Pool, Wh, Ws, bs, Wv, Wg, bg, lg, lb, blk=640):
    def body(ss_ref, p_ref, wh_r, ws_r, bs_r, wv_r, wg_r, bg_r, lg_r, lb_r,
             o_ref):
        h = ss_ref[...]
        ps, pV = _mean_from_partials(p_ref)
        s = jnp.concatenate([h[:, :64], ps], axis=-1)
        V = [jnp.concatenate([h[:, 64 + 8 * j:72 + 8 * j], pV[j]], axis=-1)
             for j in range(3)]
        ds, dV = _gvp_math(s, V, wh_r[...], ws_r[...], bs_r[...],
                           wv_r[...], wg_r[...], bg_r[...], act=jax.nn.relu)
        s2, V2 = _ln_math(h[:, :64] + ds,
                          [h[:, 64 + 8 * j:72 + 8 * j] + dV[j]
                           for j in range(3)], lg_r[...], lb_r[...])
        o_ref[...] = _pack128(s2, V2)

    ws = [Wh, Ws, bs, Wv, Wg, bg, lg, lb]
    return pl.pallas_call(
        body, grid=(SPAD // blk,),
        in_specs=[_rspec(blk, 128),
                  pl.BlockSpec((2, blk, 128), lambda i: (0, i, 0))] +
                 [_fspec(w.shape) for w in ws],
        out_specs=_rspec(blk, 128),
        out_shape=jax.ShapeDtypeStruct((SPAD, 128), F32),
        compiler_params=_TC_PARAMS,
    )(SS, Pool, *ws)


def _tc_node_sse(H2, Bc, Wh, Ws, bs, Wv, Wg, bg, lg, lb, blk=1024):
    def body(h_ref, b_ref, wh_r, ws_r, bs_r, wv_r, wg_r, bg_r, lg_r, lb_r,
             o_ref):
        h = h_ref[...]
        c = b_ref[...]
        s = jnp.concatenate([h[:, :64], c[:, :64]], axis=-1)
        V = [jnp.concatenate([h[:, 64 + 8 * j:72 + 8 * j],
                              c[:, 64 + 8 * j:72 + 8 * j]], axis=-1)
             for j in range(3)]
        ds, dV = _gvp_math(s, V, wh_r[...], ws_r[...], bs_r[...],
                           wv_r[...], wg_r[...], bg_r[...], act=jax.nn.relu)
        s2, V2 = _ln_math(h[:, :64] + ds,
                          [h[:, 64 + 8 * j:72 + 8 * j] + dV[j]
                           for j in range(3)], lg_r[...], lb_r[...])
        o_ref[...] = _pack128(s2, V2)

    ws = [Wh, Ws, bs, Wv, Wg, bg, lg, lb]
    return pl.pallas_call(
        body, grid=(NPAD // blk,),
        in_specs=[_rspec(blk, 128), _rspec(blk, 128)] +
                 [_fspec(w.shape) for w in ws],
        out_specs=_rspec(blk, 128),
        out_shape=jax.ShapeDtypeStruct((NPAD, 128), F32),
        compiler_params=_TC_PARAMS,
    )(H2, Bc, *ws)


def _tc_out(H, lg, lb, Wh, Ws, bs, blk=1024):
    def body(h_ref, lg_r, lb_r, wh_r, ws_r, bs_r, o_ref, og_ref):
        h = h_ref[...]
        s = h[:, :64]
        V = [h[:, 64 + 8 * j:72 + 8 * j] for j in range(3)]
        s, V = _ln_math(s, V, lg_r[...], lb_r[...])
        so, _ = _gvp_math(s, V, wh_r[...], ws_r[...], bs_r[...],
                          act=jax.nn.relu)
        o_ref[...] = so
        og_ref[...] = jnp.concatenate(
            [so, jnp.zeros((so.shape[0], 64), F32)], axis=-1)

    ws = [lg, lb, Wh, Ws, bs]
    return pl.pallas_call(
        body, grid=(NPAD // blk,),
        in_specs=[_rspec(blk, 128)] + [_fspec(w.shape) for w in ws],
        out_specs=[_rspec(blk, 64), _rspec(blk, 128)],
        out_shape=[jax.ShapeDtypeStruct((NPAD, 64), F32),
                   jax.ShapeDtypeStruct((NPAD, 128), F32)],
        compiler_params=_TC_PARAMS,
    )(H, *ws)


def _tc_graph_combine(Gp):
    def body(g_ref, o_ref):
        g = g_ref[...]
        o_ref[...] = (g[0] + g[1])[:, :64]

    return pl.pallas_call(
        body, grid=(1,),
        in_specs=[_fspec((2, GPAD, 128))],
        out_specs=_fspec((GPAD, 64)),
        out_shape=jax.ShapeDtypeStruct((GPAD, 64), F32),
    )(Gp)


_SC_MESH = dict(core_axis_name="c", subcore_axis_name="s")


def _sc_gather(table, idx, nrows_out):
    per_w = nrows_out // NW
    nch = per_w // CH
    fdim = table.shape[1]
    Q = 4
    nq, tail = nch // Q, nch % Q

    @functools.partial(
        pl.kernel,
        mesh=plsc.VectorSubcoreMesh(**_SC_MESH),
        out_type=jax.ShapeDtypeStruct((nrows_out, fdim), F32),
        scratch_types=[pltpu.VMEM((per_w,), jnp.int32)] +
                      [pltpu.VMEM((CH, fdim), F32)] * 4 +
                      [pltpu.SemaphoreType.DMA, pltpu.SemaphoreType.DMA],
    )
    def k(t_ref, i_ref, o_ref, idx_all, r0, r1, r2, r3, sem, semo):
        wid = lax.axis_index("s") * NCORE + lax.axis_index("c")
        base = wid * per_w
        pltpu.sync_copy(i_ref.at[pl.ds(base, per_w)], idx_all)
        rows = (r0, r1, r2, r3)

        def chunk_io(j0, nb):
            hs = []
            for b in range(nb):
                hs.append(pltpu.async_copy(
                    t_ref.at[idx_all.at[pl.ds((j0 + b) * CH, CH)]],
                    rows[b], sem))
            for h in hs:
                h.wait()
            hs = []
            for b in range(nb):
                hs.append(pltpu.async_copy(
                    rows[b], o_ref.at[pl.ds(base + (j0 + b) * CH, CH)],
                    semo))
            for h in hs:
                h.wait()

        def body(j2, carry):
            chunk_io(j2 * Q, Q)
            return carry

        lax.fori_loop(0, nq, body, 0)
        if tail:
            chunk_io(nq * Q, tail)

    return k(table, idx)


def _sc_scatter(msgs, idx, table_rows, nsrc, ngroups):
    per_w = nsrc // NW
    nch = per_w // CH
    rpt = table_rows // NSUB
    zeros = jnp.zeros((rpt, 16), F32)
    Q = 2
    nq, tail = nch // Q, nch % Q

    @functools.partial(
        pl.kernel,
        mesh=plsc.VectorSubcoreMesh(**_SC_MESH),
        out_type=jax.ShapeDtypeStruct((NCORE, table_rows, 128), F32),
        scratch_types=[pltpu.VMEM((CH,), jnp.int32),
                       pltpu.VMEM((CH,), jnp.int32),
                       pltpu.VMEM((CH, 16), F32),
                       pltpu.VMEM((CH, 16), F32),
                       pltpu.VMEM_SHARED((table_rows, 16), F32),
                       pltpu.SemaphoreType.DMA],
        compiler_params=pltpu.CompilerParams(use_tc_tiling_on_sc=False),
    )
    def k(m_ref, i_ref, z_ref, o_ref, i0, i1, v0, v1, shared, sem):
        cid = lax.axis_index("c")
        sid = lax.axis_index("s")
        wid = sid * NCORE + cid
        ic = (i0, i1)
        mv = (v0, v1)
        for g in range(ngroups):
            pltpu.sync_copy(z_ref, shared.at[pl.ds(sid * rpt, rpt)])
            plsc.subcore_barrier()

            def chunk_adds(j0, nb):
                hs = []
                for b in range(nb):
                    off = wid * per_w + (j0 + b) * CH
                    hs.append(pltpu.async_copy(
                        i_ref.at[pl.ds(off, CH)], ic[b], sem))
                    hs.append(pltpu.async_copy(
                        m_ref.at[pl.ds(off, CH), pl.ds(16 * g, 16)],
                        mv[b], sem))
                for h in hs:
                    h.wait()
                for b in range(nb):
                    pltpu.sync_copy(mv[b], shared.at[ic[b]], add=True)

            def body(j2, carry):
                chunk_adds(j2 * Q, Q)
                return carry

            lax.fori_loop(0, nq, body, 0)
            if tail:
                chunk_adds(nq * Q, tail)
            plsc.subcore_barrier()
            pltpu.sync_copy(shared.at[pl.ds(sid * rpt, rpt)],
                            o_ref.at[cid, pl.ds(sid * rpt, rpt),
                                     pl.ds(16 * g, 16)])
            plsc.subcore_barrier()

    return k(msgs, idx, zeros)


def kernel(x, x_vector_attr, edge_attr, edge_vector_attr, sse_attr,
           sse_vector_attr, params, edge_index, node_to_sse, batch):
    xs = jnp.pad(x, ((0, NPAD - N), (0, 0)))
    xv3 = jnp.pad(x_vector_attr.transpose(0, 2, 1).reshape(N, 12),
                  ((0, NPAD - N), (0, 0)))
    eas = jnp.pad(edge_attr, ((0, EPAD - E), (0, 0)))
    ev8 = jnp.pad(edge_vector_attr.reshape(E, 3), ((0, EPAD - E), (0, 5)))
    sss = jnp.pad(sse_attr, ((0, SPAD - NSSE), (0, 0)))
    ssv3 = jnp.pad(sse_vector_attr.transpose(0, 2, 1).reshape(NSSE, 12),
                   ((0, SPAD - NSSE), (0, 0)))

    epad_i = jnp.arange(EPAD - E, dtype=jnp.int32)
    npad_i = jnp.arange(NPAD - N, dtype=jnp.int32)
    src = jnp.concatenate([edge_index[0].astype(jnp.int32), epad_i % N])
    dstg = jnp.concatenate([edge_index[1].astype(jnp.int32), epad_i % N])
    dst_sc = jnp.concatenate([edge_index[1].astype(jnp.int32),
                              N + epad_i % (NPAD - N)])
    n2s = node_to_sse.astype(jnp.int32)
    n2s_g = jnp.concatenate([n2s, npad_i % NSSE])
    n2s_sc = jnp.concatenate([n2s, NSSE + npad_i % (SPAD - NSSE)])
    bat_sc = jnp.concatenate([batch.astype(jnp.int32),
                              NG + npad_i % (GPAD - NG)])

    r2 = lambda w: w.reshape(1, -1)

    def LN(p):
        return r2(p['g']), r2(p['b'])

    def GW(p):
        return (p['Wh'], p['Ws'], r2(p['bs']), p['Wv'], p['Wg'], r2(p['bg']))

    H = _tc_init(xs, xv3, *LN(params['W_v']['ln']),
                 *GW(params['W_v']['gvp']), NPAD, 1024, 64, 4)
    es_a, ev_a = _tc_edge_init(eas, ev8, *LN(params['W_e']['ln']),
                               *GW(params['W_e']['gvp']))
    SS = _tc_init(sss, ssv3, *LN(params['W_sse']['ln']),
                  *GW(params['W_sse']['gvp']), SPAD, 640, 64, 4)

    for lp in params['layers']:
        Gs = _sc_gather(H, src, EPAD)
        Gd = _sc_gather(H, dstg, EPAD)
        M = _tc_msg(Gs, Gd, es_a, ev_a, *GW(lp['msg']))
        Msum = _sc_scatter(M, dst_sc, NPAD, EPAD, 6)
        H2 = _tc_node_upd(H, Msum, *LN(lp['ln1']), *GW(lp['ff']),
                          *LN(lp['ln2']))
        Pool = _sc_scatter(H2, n2s_sc, SPAD, NPAD, 6)
        SS = _tc_sse_upd(SS, Pool, *GW(lp['sse_upd']), *LN(lp['ln_sse']))
        Bc = _sc_gather(SS, n2s_g, NPAD)
        H = _tc_node_sse(H2, Bc, *GW(lp['node_sse']), *LN(lp['ln3']))

    ne, neg = _tc_out(H, *LN(params['W_out']['ln']),
                      params['W_out']['gvp']['Wh'],
                      params['W_out']['gvp']['Ws'],
                      r2(params['W_out']['gvp']['bs']))
    Gp = _sc_scatter(neg, bat_sc, GPAD, NPAD, 4)
    ge = _tc_graph_combine(Gp)
    return ne[:N], ge[:NG]


# speedup vs baseline: 21.3046x; 1.2918x over previous
import functools

import jax
import jax.numpy as jnp
from jax import lax
from jax.experimental import pallas as pl
from jax.experimental.pallas import tpu as pltpu
from jax.experimental.pallas import tpu_sc as plsc

N = 50000
E = 800000
NSSE = 5000
NG = 50

NW = 32
NCORE = 2
NSUB = 16
CH = 128

NPAD = 53248
EPAD = 819200
SPAD = 5120
GPAD = 128

F32 = jnp.float32


def _ln_math(s, V, g, b):
    mu = jnp.mean(s, axis=-1, keepdims=True)
    var = jnp.mean((s - mu) ** 2, axis=-1, keepdims=True)
    s = (s - mu) * lax.rsqrt(var + 1e-5) * g + b
    vn2 = V[0] * V[0] + V[1] * V[1] + V[2] * V[2]
    vnorm = jnp.sqrt(jnp.mean(vn2, axis=-1, keepdims=True) + 1e-8)
    V = [v / vnorm for v in V]
    return s, V


def _gvp_math(s, V, Wh, Ws, bs, Wv=None, Wg=None, bg=None, act=None):
    Vh = [jnp.dot(v, Wh, preferred_element_type=F32) for v in V]
    vn = jnp.sqrt(Vh[0] ** 2 + Vh[1] ** 2 + Vh[2] ** 2 + 1e-8)
    so = jnp.dot(jnp.concatenate([s, vn], axis=-1), Ws,
                 preferred_element_type=F32) + bs
    Vout = None
    if Wv is not None:
        Vout = [jnp.dot(vh, Wv, preferred_element_type=F32) for vh in Vh]
        gin = act(so) if act is not None else so
        gate = jax.nn.sigmoid(jnp.dot(gin, Wg, preferred_element_type=F32) + bg)
        Vout = [v * gate for v in Vout]
    if act is not None:
        so = act(so)
    return so, Vout


def _pack128(so, Vo):
    b = so.shape[0]
    return jnp.concatenate(
        [so] + Vo + [jnp.ones((b, 1), F32), jnp.zeros((b, 39), F32)], axis=-1)


def _rspec(blk, f):
    return pl.BlockSpec((blk, f), lambda i: (i, 0))


def _fspec(shape):
    nd = len(shape)
    return pl.BlockSpec(shape, lambda i: (0,) * nd)


_TC_PARAMS = pltpu.CompilerParams(dimension_semantics=("parallel",))


def _tc_init(s_in, v_in, lg, lb, Wh, Ws, bs, Wv, Wg, bg, nrows, blk, si, vi):
    def body(s_ref, v_ref, lg_r, lb_r, wh_r, ws_r, bs_r, wv_r, wg_r, bg_r,
             o_ref):
        s = s_ref[...].T
        vt = v_ref[...].T
        V = [vt[:, j * vi:(j + 1) * vi] for j in range(3)]
        s, V = _ln_math(s, V, lg_r[...], lb_r[...])
        so, Vo = _gvp_math(s, V, wh_r[...], ws_r[...], bs_r[...],
                           wv_r[...], wg_r[...], bg_r[...], act=None)
        o_ref[...] = _pack128(so, Vo)

    ws = [lg, lb, Wh, Ws, bs, Wv, Wg, bg]
    return pl.pallas_call(
        body, grid=(nrows // blk,),
        in_specs=[pl.BlockSpec((si, blk), lambda i: (0, i)),
                  pl.BlockSpec((16, blk), lambda i: (0, i))] +
                 [_fspec(w.shape) for w in ws],
        out_specs=_rspec(blk, 128),
        out_shape=jax.ShapeDtypeStruct((nrows, 128), F32),
        compiler_params=_TC_PARAMS,
    )(s_in, v_in, *ws)


def _tc_edge_init(ea, ev8, lg, lb, Wh, Ws, bs, Wv, Wg, bg, blk=1024):
    def body(a_ref, v_ref, lg_r, lb_r, wh_r, ws_r, bs_r, wv_r, wg_r, bg_r,
             so_ref, vo_ref):
        s = a_ref[...].T
        vt = v_ref[...].T
        V = [vt[:, j:j + 1] for j in range(3)]
        s, V = _ln_math(s, V, lg_r[...], lb_r[...])
        so, Vo = _gvp_math(s, V, wh_r[...], ws_r[...], bs_r[...],
                           wv_r[...], wg_r[...], bg_r[...], act=None)
        so_ref[...] = so
        vo_ref[...] = jnp.concatenate(
            Vo + [jnp.zeros((so.shape[0], 5), F32)], axis=-1)

    ws = [lg, lb, Wh, Ws, bs, Wv, Wg, bg]
    return pl.pallas_call(
        body, grid=(EPAD // blk,),
        in_specs=[pl.BlockSpec((32, blk), lambda i: (0, i)),
                  pl.BlockSpec((8, blk), lambda i: (0, i))] +
                 [_fspec(w.shape) for w in ws],
        out_specs=[_rspec(blk, 32), _rspec(blk, 8)],
        out_shape=[jax.ShapeDtypeStruct((EPAD, 32), F32),
                   jax.ShapeDtypeStruct((EPAD, 8), F32)],
        compiler_params=_TC_PARAMS,
    )(ea, ev8, *ws)


def _tc_msg(gs, gd, es, ev8, Wh, Ws, bs, Wv, Wg, bg, blk=1024):
    def body(gs_ref, gd_ref, es_ref, ev_ref, wh_r, ws_r, bs_r, wv_r, wg_r,
             bg_r, o_ref):
        a = gs_ref[...]
        b = gd_ref[...]
        s = jnp.concatenate([a[:, :64], es_ref[...], b[:, :64]], axis=-1)
        ev = ev_ref[...]
        V = [jnp.concatenate([a[:, 64 + 8 * j:72 + 8 * j], ev[:, j:j + 1],
                              b[:, 64 + 8 * j:72 + 8 * j]], axis=-1)
             for j in range(3)]
        so, Vo = _gvp_math(s, V, wh_r[...], ws_r[...], bs_r[...],
                           wv_r[...], wg_r[...], bg_r[...], act=jax.nn.relu)
        o_ref[...] = _pack128(so, Vo)

    ws = [Wh, Ws, bs, Wv, Wg, bg]
    return pl.pallas_call(
        body, grid=(EPAD // blk,),
        in_specs=[_rspec(blk, 128), _rspec(blk, 128), _rspec(blk, 32),
                  _rspec(blk, 8)] + [_fspec(w.shape) for w in ws],
        out_specs=_rspec(blk, 128),
        out_shape=jax.ShapeDtypeStruct((EPAD, 128), F32),
        compiler_params=_TC_PARAMS,
    )(gs, gd, es, ev8, *ws)


def _mean_from_partials(m_ref):
    m = m_ref[...]
    m = m[0] + m[1]
    cnt = jnp.maximum(m[:, 88:89], 1.0)
    ms = m[:, :64] / cnt
    mV = [m[:, 64 + 8 * j:72 + 8 * j] / cnt for j in range(3)]
    return ms, mV


def _tc_node_upd(H, Msum, l1g, l1b, Wh, Ws, bs, Wv, Wg, bg, l2g, l2b,
                 blk=1024):
    def body(h_ref, m_ref, l1g_r, l1b_r, wh_r, ws_r, bs_r, wv_r, wg_r, bg_r,
             l2g_r, l2b_r, o_ref):
        h = h_ref[...]
        ms, mV = _mean_from_partials(m_ref)
        s = h[:, :64] + ms
        V = [h[:, 64 + 8 * j:72 + 8 * j] + mV[j] for j in range(3)]
        s, V = _ln_math(s, V, l1g_r[...], l1b_r[...])
        ds, dV = _gvp_math(s, V, wh_r[...], ws_r[...], bs_r[...],
                           wv_r[...], wg_r[...], bg_r[...], act=jax.nn.relu)
        s2, V2 = _ln_math(s + ds, [V[j] + dV[j] for j in range(3)],
                          l2g_r[...], l2b_r[...])
        o_ref[...] = _pack128(s2, V2)

    ws = [l1g, l1b, Wh, Ws, bs, Wv, Wg, bg, l2g, l2b]
    return pl.pallas_call(
        body, grid=(NPAD // blk,),
        in_specs=[_rspec(blk, 128),
                  pl.BlockSpec((2, blk, 128), lambda i: (0, i, 0))] +
                 [_fspec(w.shape) for w in ws],
        out_specs=_rspec(blk, 128),
        out_shape=jax.ShapeDtypeStruct((NPAD, 128), F32),
        compiler_params=_TC_PARAMS,
    )(H, Msum, *ws)


def _tc_sse_upd(SS, Pool, Wh, Ws, bs, Wv, Wg, bg, lg, lb, blk=640):
    def body(ss_ref, p_ref, wh_r, ws_r, bs_r, wv_r, wg_r, bg_r, lg_r, lb_r,
             o_ref):
        h = ss_ref[...]
        ps, pV = _mean_from_partials(p_ref)
        s = jnp.concatenate([h[:, :64], ps], axis=-1)
        V = [jnp.concatenate([h[:, 64 + 8 * j:72 + 8 * j], pV[j]], axis=-1)
             for j in range(3)]
        ds, dV = _gvp_math(s, V, wh_r[...], ws_r[...], bs_r[...],
                           wv_r[...], wg_r[...], bg_r[...], act=jax.nn.relu)
        s2, V2 = _ln_math(h[:, :64] + ds,
                          [h[:, 64 + 8 * j:72 + 8 * j] + dV[j]
                           for j in range(3)], lg_r[...], lb_r[...])
        o_ref[...] = _pack128(s2, V2)

    ws = [Wh, Ws, bs, Wv, Wg, bg, lg, lb]
    return pl.pallas_call(
        body, grid=(SPAD // blk,),
        in_specs=[_rspec(blk, 128),
                  pl.BlockSpec((2, blk, 128), lambda i: (0, i, 0))] +
                 [_fspec(w.shape) for w in ws],
        out_specs=_rspec(blk, 128),
        out_shape=jax.ShapeDtypeStruct((SPAD, 128), F32),
        compiler_params=_TC_PARAMS,
    )(SS, Pool, *ws)


def _tc_node_sse(H2, Bc, Wh, Ws, bs, Wv, Wg, bg, lg, lb, blk=1024):
    def body(h_ref, b_ref, wh_r, ws_r, bs_r, wv_r, wg_r, bg_r, lg_r, lb_r,
             o_ref):
        h = h_ref[...]
        c = b_ref[...]
        s = jnp.concatenate([h[:, :64], c[:, :64]], axis=-1)
        V = [jnp.concatenate([h[:, 64 + 8 * j:72 + 8 * j],
                              c[:, 64 + 8 * j:72 + 8 * j]], axis=-1)
             for j in range(3)]
        ds, dV = _gvp_math(s, V, wh_r[...], ws_r[...], bs_r[...],
                           wv_r[...], wg_r[...], bg_r[...], act=jax.nn.relu)
        s2, V2 = _ln_math(h[:, :64] + ds,
                          [h[:, 64 + 8 * j:72 + 8 * j] + dV[j]
                           for j in range(3)], lg_r[...], lb_r[...])
        o_ref[...] = _pack128(s2, V2)

    ws = [Wh, Ws, bs, Wv, Wg, bg, lg, lb]
    return pl.pallas_call(
        body, grid=(NPAD // blk,),
        in_specs=[_rspec(blk, 128), _rspec(blk, 128)] +
                 [_fspec(w.shape) for w in ws],
        out_specs=_rspec(blk, 128),
        out_shape=jax.ShapeDtypeStruct((NPAD, 128), F32),
        compiler_params=_TC_PARAMS,
    )(H2, Bc, *ws)


def _tc_out(H, lg, lb, Wh, Ws, bs, blk=1024):
    def body(h_ref, lg_r, lb_r, wh_r, ws_r, bs_r, o_ref, og_ref):
        h = h_ref[...]
        s = h[:, :64]
        V = [h[:, 64 + 8 * j:72 + 8 * j] for j in range(3)]
        s, V = _ln_math(s, V, lg_r[...], lb_r[...])
        so, _ = _gvp_math(s, V, wh_r[...], ws_r[...], bs_r[...],
                          act=jax.nn.relu)
        o_ref[...] = so
        og_ref[...] = jnp.concatenate(
            [so, jnp.zeros((so.shape[0], 64), F32)], axis=-1)

    ws = [lg, lb, Wh, Ws, bs]
    return pl.pallas_call(
        body, grid=(NPAD // blk,),
        in_specs=[_rspec(blk, 128)] + [_fspec(w.shape) for w in ws],
        out_specs=[_rspec(blk, 64), _rspec(blk, 128)],
        out_shape=[jax.ShapeDtypeStruct((NPAD, 64), F32),
                   jax.ShapeDtypeStruct((NPAD, 128), F32)],
        compiler_params=_TC_PARAMS,
    )(H, *ws)


def _tc_graph_combine(Gp):
    def body(g_ref, o_ref):
        g = g_ref[...]
        o_ref[...] = (g[0] + g[1])[:, :64]

    return pl.pallas_call(
        body, grid=(1,),
        in_specs=[_fspec((2, GPAD, 128))],
        out_specs=_fspec((GPAD, 64)),
        out_shape=jax.ShapeDtypeStruct((GPAD, 64), F32),
    )(Gp)


_SC_MESH = dict(core_axis_name="c", subcore_axis_name="s")


def _sc_gather(table, idx, nrows_out):
    per_w = nrows_out // NW
    nch = per_w // CH
    fdim = table.shape[1]
    Q = 4
    nq, tail = nch // Q, nch % Q

    @functools.partial(
        pl.kernel,
        mesh=plsc.VectorSubcoreMesh(**_SC_MESH),
        out_type=jax.ShapeDtypeStruct((nrows_out, fdim), F32),
        scratch_types=[pltpu.VMEM((per_w,), jnp.int32)] +
                      [pltpu.VMEM((CH, fdim), F32)] * 4 +
                      [pltpu.SemaphoreType.DMA, pltpu.SemaphoreType.DMA],
    )
    def k(t_ref, i_ref, o_ref, idx_all, r0, r1, r2, r3, sem, semo):
        wid = lax.axis_index("s") * NCORE + lax.axis_index("c")
        base = wid * per_w
        pltpu.sync_copy(i_ref.at[pl.ds(base, per_w)], idx_all)
        rows = (r0, r1, r2, r3)

        def chunk_io(j0, nb):
            hs = []
            for b in range(nb):
                hs.append(pltpu.async_copy(
                    t_ref.at[idx_all.at[pl.ds((j0 + b) * CH, CH)]],
                    rows[b], sem))
            for h in hs:
                h.wait()
            hs = []
            for b in range(nb):
                hs.append(pltpu.async_copy(
                    rows[b], o_ref.at[pl.ds(base + (j0 + b) * CH, CH)],
                    semo))
            for h in hs:
                h.wait()

        def body(j2, carry):
            chunk_io(j2 * Q, Q)
            return carry

        lax.fori_loop(0, nq, body, 0)
        if tail:
            chunk_io(nq * Q, tail)

    return k(table, idx)


def _sc_scatter(msgs, idx, table_rows, nsrc, ngroups):
    per_w = nsrc // NW
    nch = per_w // CH
    rpt = table_rows // NSUB
    zeros = jnp.zeros((rpt, 16), F32)
    Q = 2
    nq, tail = nch // Q, nch % Q

    @functools.partial(
        pl.kernel,
        mesh=plsc.VectorSubcoreMesh(**_SC_MESH),
        out_type=jax.ShapeDtypeStruct((NCORE, table_rows, 128), F32),
        scratch_types=[pltpu.VMEM((CH,), jnp.int32),
                       pltpu.VMEM((CH,), jnp.int32),
                       pltpu.VMEM((CH, 16), F32),
                       pltpu.VMEM((CH, 16), F32),
                       pltpu.VMEM_SHARED((table_rows, 16), F32),
                       pltpu.SemaphoreType.DMA],
        compiler_params=pltpu.CompilerParams(use_tc_tiling_on_sc=False),
    )
    def k(m_ref, i_ref, z_ref, o_ref, i0, i1, v0, v1, shared, sem):
        cid = lax.axis_index("c")
        sid = lax.axis_index("s")
        wid = sid * NCORE + cid
        ic = (i0, i1)
        mv = (v0, v1)
        for g in range(ngroups):
            pltpu.sync_copy(z_ref, shared.at[pl.ds(sid * rpt, rpt)])
            plsc.subcore_barrier()

            def chunk_adds(j0, nb):
                hs = []
                for b in range(nb):
                    off = wid * per_w + (j0 + b) * CH
                    hs.append(pltpu.async_copy(
                        i_ref.at[pl.ds(off, CH)], ic[b], sem))
                    hs.append(pltpu.async_copy(
                        m_ref.at[pl.ds(off, CH), pl.ds(16 * g, 16)],
                        mv[b], sem))
                for h in hs:
                    h.wait()
                for b in range(nb):
                    pltpu.sync_copy(mv[b], shared.at[ic[b]], add=True)

            def body(j2, carry):
                chunk_adds(j2 * Q, Q)
                return carry

            lax.fori_loop(0, nq, body, 0)
            if tail:
                chunk_adds(nq * Q, tail)
            plsc.subcore_barrier()
            pltpu.sync_copy(shared.at[pl.ds(sid * rpt, rpt)],
                            o_ref.at[cid, pl.ds(sid * rpt, rpt),
                                     pl.ds(16 * g, 16)])
            plsc.subcore_barrier()

    return k(msgs, idx, zeros)


def kernel(x, x_vector_attr, edge_attr, edge_vector_attr, sse_attr,
           sse_vector_attr, params, edge_index, node_to_sse, batch):
    xs = jnp.pad(x.T, ((0, 0), (0, NPAD - N)))
    xv3 = jnp.pad(x_vector_attr.transpose(2, 1, 0).reshape(12, N),
                  ((0, 4), (0, NPAD - N)))
    eas = jnp.pad(edge_attr.T, ((0, 0), (0, EPAD - E)))
    ev8 = jnp.pad(edge_vector_attr.transpose(2, 1, 0).reshape(3, E),
                  ((0, 5), (0, EPAD - E)))
    sss = jnp.pad(sse_attr.T, ((0, 0), (0, SPAD - NSSE)))
    ssv3 = jnp.pad(sse_vector_attr.transpose(2, 1, 0).reshape(12, NSSE),
                   ((0, 4), (0, SPAD - NSSE)))

    epad_i = jnp.arange(EPAD - E, dtype=jnp.int32)
    npad_i = jnp.arange(NPAD - N, dtype=jnp.int32)
    src = jnp.concatenate([edge_index[0].astype(jnp.int32), epad_i % N])
    dstg = jnp.concatenate([edge_index[1].astype(jnp.int32), epad_i % N])
    dst_sc = jnp.concatenate([edge_index[1].astype(jnp.int32),
                              N + epad_i % (NPAD - N)])
    n2s = node_to_sse.astype(jnp.int32)
    n2s_g = jnp.concatenate([n2s, npad_i % NSSE])
    n2s_sc = jnp.concatenate([n2s, NSSE + npad_i % (SPAD - NSSE)])
    bat_sc = jnp.concatenate([batch.astype(jnp.int32),
                              NG + npad_i % (GPAD - NG)])

    r2 = lambda w: w.reshape(1, -1)

    def LN(p):
        return r2(p['g']), r2(p['b'])

    def GW(p):
        return (p['Wh'], p['Ws'], r2(p['bs']), p['Wv'], p['Wg'], r2(p['bg']))

    H = _tc_init(xs, xv3, *LN(params['W_v']['ln']),
                 *GW(params['W_v']['gvp']), NPAD, 1024, 64, 4)
    es_a, ev_a = _tc_edge_init(eas, ev8, *LN(params['W_e']['ln']),
                               *GW(params['W_e']['gvp']))
    SS = _tc_init(sss, ssv3, *LN(params['W_sse']['ln']),
                  *GW(params['W_sse']['gvp']), SPAD, 640, 64, 4)

    for lp in params['layers']:
        Gs = _sc_gather(H, src, EPAD)
        Gd = _sc_gather(H, dstg, EPAD)
        M = _tc_msg(Gs, Gd, es_a, ev_a, *GW(lp['msg']))
        Msum = _sc_scatter(M, dst_sc, NPAD, EPAD, 6)
        H2 = _tc_node_upd(H, Msum, *LN(lp['ln1']), *GW(lp['ff']),
                          *LN(lp['ln2']))
        Pool = _sc_scatter(H2, n2s_sc, SPAD, NPAD, 6)
        SS = _tc_sse_upd(SS, Pool, *GW(lp['sse_upd']), *LN(lp['ln_sse']))
        Bc = _sc_gather(SS, n2s_g, NPAD)
        H = _tc_node_sse(H2, Bc, *GW(lp['node_sse']), *LN(lp['ln3']))

    ne, neg = _tc_out(H, *LN(params['W_out']['ln']),
                      params['W_out']['gvp']['Wh'],
                      params['W_out']['gvp']['Ws'],
                      r2(params['W_out']['gvp']['bs']))
    Gp = _sc_scatter(neg, bat_sc, GPAD, NPAD, 4)
    ge = _tc_graph_combine(Gp)
    return ne[:N], ge[:NG]


# speedup vs baseline: 22.4291x; 1.0528x over previous
import functools

import jax
import jax.numpy as jnp
from jax import lax
from jax.experimental import pallas as pl
from jax.experimental.pallas import tpu as pltpu
from jax.experimental.pallas import tpu_sc as plsc

N = 50000
E = 800000
NSSE = 5000
NG = 50

NW = 32
NCORE = 2
NSUB = 16
CH = 128

NPAD = 53248
EPAD = 819200
SPAD = 5120
GPAD = 128

F32 = jnp.float32


def _ln_math(s, V, g, b):
    mu = jnp.mean(s, axis=-1, keepdims=True)
    var = jnp.mean((s - mu) ** 2, axis=-1, keepdims=True)
    s = (s - mu) * lax.rsqrt(var + 1e-5) * g + b
    vn2 = V[0] * V[0] + V[1] * V[1] + V[2] * V[2]
    vnorm = jnp.sqrt(jnp.mean(vn2, axis=-1, keepdims=True) + 1e-8)
    V = [v / vnorm for v in V]
    return s, V


def _gvp_math(s, V, Wh, Ws, bs, Wv=None, Wg=None, bg=None, act=None):
    Vh = [jnp.dot(v, Wh, preferred_element_type=F32) for v in V]
    vn = jnp.sqrt(Vh[0] ** 2 + Vh[1] ** 2 + Vh[2] ** 2 + 1e-8)
    so = jnp.dot(jnp.concatenate([s, vn], axis=-1), Ws,
                 preferred_element_type=F32) + bs
    Vout = None
    if Wv is not None:
        Vout = [jnp.dot(vh, Wv, preferred_element_type=F32) for vh in Vh]
        gin = act(so) if act is not None else so
        gate = jax.nn.sigmoid(jnp.dot(gin, Wg, preferred_element_type=F32) + bg)
        Vout = [v * gate for v in Vout]
    if act is not None:
        so = act(so)
    return so, Vout


def _pack128(so, Vo):
    b = so.shape[0]
    return jnp.concatenate(
        [so] + Vo + [jnp.ones((b, 1), F32), jnp.zeros((b, 39), F32)], axis=-1)


def _rspec(blk, f):
    return pl.BlockSpec((blk, f), lambda i: (i, 0))


def _fspec(shape):
    nd = len(shape)
    return pl.BlockSpec(shape, lambda i: (0,) * nd)


_TC_PARAMS = pltpu.CompilerParams(dimension_semantics=("parallel",))


def _tc_init(s_in, v_in, lg, lb, Wh, Ws, bs, Wv, Wg, bg, nrows, blk, si, vi):
    def body(s_ref, v_ref, lg_r, lb_r, wh_r, ws_r, bs_r, wv_r, wg_r, bg_r,
             o_ref):
        s = s_ref[...].T
        vt = v_ref[...].T
        V = [vt[:, j * vi:(j + 1) * vi] for j in range(3)]
        s, V = _ln_math(s, V, lg_r[...], lb_r[...])
        so, Vo = _gvp_math(s, V, wh_r[...], ws_r[...], bs_r[...],
                           wv_r[...], wg_r[...], bg_r[...], act=None)
        o_ref[...] = _pack128(so, Vo)

    ws = [lg, lb, Wh, Ws, bs, Wv, Wg, bg]
    return pl.pallas_call(
        body, grid=(nrows // blk,),
        in_specs=[pl.BlockSpec((si, blk), lambda i: (0, i)),
                  pl.BlockSpec((16, blk), lambda i: (0, i))] +
                 [_fspec(w.shape) for w in ws],
        out_specs=_rspec(blk, 128),
        out_shape=jax.ShapeDtypeStruct((nrows, 128), F32),
        compiler_params=_TC_PARAMS,
    )(s_in, v_in, *ws)


def _tc_edge_init(ea, ev8, lg, lb, Wh, Ws, bs, Wv, Wg, bg, blk=1024):
    def body(a_ref, v_ref, lg_r, lb_r, wh_r, ws_r, bs_r, wv_r, wg_r, bg_r,
             so_ref, vo_ref):
        s = a_ref[...].T
        vt = v_ref[...].T
        V = [vt[:, j:j + 1] for j in range(3)]
        s, V = _ln_math(s, V, lg_r[...], lb_r[...])
        so, Vo = _gvp_math(s, V, wh_r[...], ws_r[...], bs_r[...],
                           wv_r[...], wg_r[...], bg_r[...], act=None)
        so_ref[...] = so
        vo_ref[...] = jnp.concatenate(
            Vo + [jnp.zeros((so.shape[0], 5), F32)], axis=-1)

    ws = [lg, lb, Wh, Ws, bs, Wv, Wg, bg]
    return pl.pallas_call(
        body, grid=(EPAD // blk,),
        in_specs=[pl.BlockSpec((32, blk), lambda i: (0, i)),
                  pl.BlockSpec((8, blk), lambda i: (0, i))] +
                 [_fspec(w.shape) for w in ws],
        out_specs=[_rspec(blk, 32), _rspec(blk, 8)],
        out_shape=[jax.ShapeDtypeStruct((EPAD, 32), F32),
                   jax.ShapeDtypeStruct((EPAD, 8), F32)],
        compiler_params=_TC_PARAMS,
    )(ea, ev8, *ws)


def _tc_msg(gs, gd, es, ev8, Wh, Ws, bs, Wv, Wg, bg, blk=1024):
    def body(gs_ref, gd_ref, es_ref, ev_ref, wh_r, ws_r, bs_r, wv_r, wg_r,
             bg_r, o_ref):
        a = gs_ref[...]
        b = gd_ref[...]
        s = jnp.concatenate([a[:, :64], es_ref[...], b[:, :64]], axis=-1)
        ev = ev_ref[...]
        V = [jnp.concatenate([a[:, 64 + 8 * j:72 + 8 * j], ev[:, j:j + 1],
                              b[:, 64 + 8 * j:72 + 8 * j]], axis=-1)
             for j in range(3)]
        so, Vo = _gvp_math(s, V, wh_r[...], ws_r[...], bs_r[...],
                           wv_r[...], wg_r[...], bg_r[...], act=jax.nn.relu)
        o_ref[...] = _pack128(so, Vo)

    ws = [Wh, Ws, bs, Wv, Wg, bg]
    return pl.pallas_call(
        body, grid=(EPAD // blk,),
        in_specs=[_rspec(blk, 128), _rspec(blk, 128), _rspec(blk, 32),
                  _rspec(blk, 8)] + [_fspec(w.shape) for w in ws],
        out_specs=_rspec(blk, 128),
        out_shape=jax.ShapeDtypeStruct((EPAD, 128), F32),
        compiler_params=_TC_PARAMS,
    )(gs, gd, es, ev8, *ws)


def _mean_from_partials(m_ref):
    m = m_ref[...]
    m = m[0] + m[1]
    cnt = jnp.maximum(m[:, 88:89], 1.0)
    ms = m[:, :64] / cnt
    mV = [m[:, 64 + 8 * j:72 + 8 * j] / cnt for j in range(3)]
    return ms, mV


def _tc_node_upd(H, Msum, l1g, l1b, Wh, Ws, bs, Wv, Wg, bg, l2g, l2b,
                 blk=1024):
    def body(h_ref, m_ref, l1g_r, l1b_r, wh_r, ws_r, bs_r, wv_r, wg_r, bg_r,
             l2g_r, l2b_r, o_ref):
        h = h_ref[...]
        ms, mV = _mean_from_partials(m_ref)
        s = h[:, :64] + ms
        V = [h[:, 64 + 8 * j:72 + 8 * j] + mV[j] for j in range(3)]
        s, V = _ln_math(s, V, l1g_r[...], l1b_r[...])
        ds, dV = _gvp_math(s, V, wh_r[...], ws_r[...], bs_r[...],
                           wv_r[...], wg_r[...], bg_r[...], act=jax.nn.relu)
        s2, V2 = _ln_math(s + ds, [V[j] + dV[j] for j in range(3)],
                          l2g_r[...], l2b_r[...])
        o_ref[...] = _pack128(s2, V2)

    ws = [l1g, l1b, Wh, Ws, bs, Wv, Wg, bg, l2g, l2b]
    return pl.pallas_call(
        body, grid=(NPAD // blk,),
        in_specs=[_rspec(blk, 128),
                  pl.BlockSpec((2, blk, 128), lambda i: (0, i, 0))] +
                 [_fspec(w.shape) for w in ws],
        out_specs=_rspec(blk, 128),
        out_shape=jax.ShapeDtypeStruct((NPAD, 128), F32),
        compiler_params=_TC_PARAMS,
    )(H, Msum, *ws)


def _tc_sse_upd(SS, Pool, Wh, Ws, bs, Wv, Wg, bg, lg, lb, blk=640):
    def body(ss_ref, p_ref, wh_r, ws_r, bs_r, wv_r, wg_r, bg_r, lg_r, lb_r,
             o_ref):
        h = ss_ref[...]
        ps, pV = _mean_from_partials(p_ref)
        s = jnp.concatenate([h[:, :64], ps], axis=-1)
        V = [jnp.concatenate([h[:, 64 + 8 * j:72 + 8 * j], pV[j]], axis=-1)
             for j in range(3)]
        ds, dV = _gvp_math(s, V, wh_r[...], ws_r[...], bs_r[...],
                           wv_r[...], wg_r[...], bg_r[...], act=jax.nn.relu)
        s2, V2 = _ln_math(h[:, :64] + ds,
                          [h[:, 64 + 8 * j:72 + 8 * j] + dV[j]
                           for j in range(3)], lg_r[...], lb_r[...])
        o_ref[...] = _pack128(s2, V2)

    ws = [Wh, Ws, bs, Wv, Wg, bg, lg, lb]
    return pl.pallas_call(
        body, grid=(SPAD // blk,),
        in_specs=[_rspec(blk, 128),
                  pl.BlockSpec((2, blk, 128), lambda i: (0, i, 0))] +
                 [_fspec(w.shape) for w in ws],
        out_specs=_rspec(blk, 128),
        out_shape=jax.ShapeDtypeStruct((SPAD, 128), F32),
        compiler_params=_TC_PARAMS,
    )(SS, Pool, *ws)


def _tc_node_sse(H2, Bc, Wh, Ws, bs, Wv, Wg, bg, lg, lb, blk=1024):
    def body(h_ref, b_ref, wh_r, ws_r, bs_r, wv_r, wg_r, bg_r, lg_r, lb_r,
             o_ref):
        h = h_ref[...]
        c = b_ref[...]
        s = jnp.concatenate([h[:, :64], c[:, :64]], axis=-1)
        V = [jnp.concatenate([h[:, 64 + 8 * j:72 + 8 * j],
                              c[:, 64 + 8 * j:72 + 8 * j]], axis=-1)
             for j in range(3)]
        ds, dV = _gvp_math(s, V, wh_r[...], ws_r[...], bs_r[...],
                           wv_r[...], wg_r[...], bg_r[...], act=jax.nn.relu)
        s2, V2 = _ln_math(h[:, :64] + ds,
                          [h[:, 64 + 8 * j:72 + 8 * j] + dV[j]
                           for j in range(3)], lg_r[...], lb_r[...])
        o_ref[...] = _pack128(s2, V2)

    ws = [Wh, Ws, bs, Wv, Wg, bg, lg, lb]
    return pl.pallas_call(
        body, grid=(NPAD // blk,),
        in_specs=[_rspec(blk, 128), _rspec(blk, 128)] +
                 [_fspec(w.shape) for w in ws],
        out_specs=_rspec(blk, 128),
        out_shape=jax.ShapeDtypeStruct((NPAD, 128), F32),
        compiler_params=_TC_PARAMS,
    )(H2, Bc, *ws)


def _tc_out(H, lg, lb, Wh, Ws, bs, blk=1024):
    def body(h_ref, lg_r, lb_r, wh_r, ws_r, bs_r, o_ref, og_ref):
        h = h_ref[...]
        s = h[:, :64]
        V = [h[:, 64 + 8 * j:72 + 8 * j] for j in range(3)]
        s, V = _ln_math(s, V, lg_r[...], lb_r[...])
        so, _ = _gvp_math(s, V, wh_r[...], ws_r[...], bs_r[...],
                          act=jax.nn.relu)
        o_ref[...] = so
        og_ref[...] = jnp.concatenate(
            [so, jnp.zeros((so.shape[0], 64), F32)], axis=-1)

    ws = [lg, lb, Wh, Ws, bs]
    return pl.pallas_call(
        body, grid=(NPAD // blk,),
        in_specs=[_rspec(blk, 128)] + [_fspec(w.shape) for w in ws],
        out_specs=[_rspec(blk, 64), _rspec(blk, 128)],
        out_shape=[jax.ShapeDtypeStruct((NPAD, 64), F32),
                   jax.ShapeDtypeStruct((NPAD, 128), F32)],
        compiler_params=_TC_PARAMS,
    )(H, *ws)


def _tc_graph_combine(Gp):
    def body(g_ref, o_ref):
        g = g_ref[...]
        o_ref[...] = (g[0] + g[1])[:, :64]

    return pl.pallas_call(
        body, grid=(1,),
        in_specs=[_fspec((2, GPAD, 128))],
        out_specs=_fspec((GPAD, 64)),
        out_shape=jax.ShapeDtypeStruct((GPAD, 64), F32),
    )(Gp)


_SC_MESH = dict(core_axis_name="c", subcore_axis_name="s")


def _sc_gather(table, idx, nrows_out):
    per_w = nrows_out // NW
    nch = per_w // CH
    fdim = table.shape[1]
    Q = 4
    nq, tail = nch // Q, nch % Q

    @functools.partial(
        pl.kernel,
        mesh=plsc.VectorSubcoreMesh(**_SC_MESH),
        out_type=jax.ShapeDtypeStruct((nrows_out, fdim), F32),
        scratch_types=[pltpu.VMEM((per_w,), jnp.int32)] +
                      [pltpu.VMEM((CH, fdim), F32)] * 4 +
                      [pltpu.SemaphoreType.DMA, pltpu.SemaphoreType.DMA],
    )
    def k(t_ref, i_ref, o_ref, idx_all, r0, r1, r2, r3, sem, semo):
        wid = lax.axis_index("s") * NCORE + lax.axis_index("c")
        base = wid * per_w
        pltpu.sync_copy(i_ref.at[pl.ds(base, per_w)], idx_all)
        rows = (r0, r1, r2, r3)

        def chunk_io(j0, nb):
            hs = []
            for b in range(nb):
                hs.append(pltpu.async_copy(
                    t_ref.at[idx_all.at[pl.ds((j0 + b) * CH, CH)]],
                    rows[b], sem))
            for h in hs:
                h.wait()
            hs = []
            for b in range(nb):
                hs.append(pltpu.async_copy(
                    rows[b], o_ref.at[pl.ds(base + (j0 + b) * CH, CH)],
                    semo))
            for h in hs:
                h.wait()

        def body(j2, carry):
            chunk_io(j2 * Q, Q)
            return carry

        lax.fori_loop(0, nq, body, 0)
        if tail:
            chunk_io(nq * Q, tail)

    return k(table, idx)


def _sc_scatter(msgs, idx, table_rows, nsrc, ngroups):
    per_w = nsrc // NW
    nch = per_w // CH
    rpt = table_rows // NSUB
    zeros = jnp.zeros((rpt, 16), F32)
    Q = 4
    nq, tail = nch // Q, nch % Q

    @functools.partial(
        pl.kernel,
        mesh=plsc.VectorSubcoreMesh(**_SC_MESH),
        out_type=jax.ShapeDtypeStruct((NCORE, table_rows, 128), F32),
        scratch_types=[pltpu.VMEM((CH,), jnp.int32)] * 4 +
                      [pltpu.VMEM((CH, 16), F32)] * 4 +
                      [pltpu.VMEM_SHARED((table_rows, 16), F32),
                       pltpu.SemaphoreType.DMA, pltpu.SemaphoreType.DMA],
        compiler_params=pltpu.CompilerParams(use_tc_tiling_on_sc=False),
    )
    def k(m_ref, i_ref, z_ref, o_ref, i0, i1, i2, i3, v0, v1, v2, v3,
          shared, sem, sema):
        cid = lax.axis_index("c")
        sid = lax.axis_index("s")
        wid = sid * NCORE + cid
        ic = (i0, i1, i2, i3)
        mv = (v0, v1, v2, v3)
        for g in range(ngroups):
            pltpu.sync_copy(z_ref, shared.at[pl.ds(sid * rpt, rpt)])
            plsc.subcore_barrier()

            def chunk_adds(j0, nb):
                hs = []
                for b in range(nb):
                    off = wid * per_w + (j0 + b) * CH
                    hs.append(pltpu.async_copy(
                        i_ref.at[pl.ds(off, CH)], ic[b], sem))
                    hs.append(pltpu.async_copy(
                        m_ref.at[pl.ds(off, CH), pl.ds(16 * g, 16)],
                        mv[b], sem))
                for h in hs:
                    h.wait()
                hs = []
                for b in range(nb):
                    hs.append(pltpu.async_copy(
                        mv[b], shared.at[ic[b]], sema, add=True))
                for h in hs:
                    h.wait()

            def body(j2, carry):
                chunk_adds(j2 * Q, Q)
                return carry

            lax.fori_loop(0, nq, body, 0)
            if tail:
                chunk_adds(nq * Q, tail)
            plsc.subcore_barrier()
            pltpu.sync_copy(shared.at[pl.ds(sid * rpt, rpt)],
                            o_ref.at[cid, pl.ds(sid * rpt, rpt),
                                     pl.ds(16 * g, 16)])
            plsc.subcore_barrier()

    return k(msgs, idx, zeros)


def kernel(x, x_vector_attr, edge_attr, edge_vector_attr, sse_attr,
           sse_vector_attr, params, edge_index, node_to_sse, batch):
    xs = jnp.pad(x.T, ((0, 0), (0, NPAD - N)))
    xv3 = jnp.pad(x_vector_attr.transpose(2, 1, 0).reshape(12, N),
                  ((0, 4), (0, NPAD - N)))
    eas = jnp.pad(edge_attr.T, ((0, 0), (0, EPAD - E)))
    ev8 = jnp.pad(edge_vector_attr.transpose(2, 1, 0).reshape(3, E),
                  ((0, 5), (0, EPAD - E)))
    sss = jnp.pad(sse_attr.T, ((0, 0), (0, SPAD - NSSE)))
    ssv3 = jnp.pad(sse_vector_attr.transpose(2, 1, 0).reshape(12, NSSE),
                   ((0, 4), (0, SPAD - NSSE)))

    epad_i = jnp.arange(EPAD - E, dtype=jnp.int32)
    npad_i = jnp.arange(NPAD - N, dtype=jnp.int32)
    src = jnp.concatenate([edge_index[0].astype(jnp.int32), epad_i % N])
    dstg = jnp.concatenate([edge_index[1].astype(jnp.int32), epad_i % N])
    dst_sc = jnp.concatenate([edge_index[1].astype(jnp.int32),
                              N + epad_i % (NPAD - N)])
    n2s = node_to_sse.astype(jnp.int32)
    n2s_g = jnp.concatenate([n2s, npad_i % NSSE])
    n2s_sc = jnp.concatenate([n2s, NSSE + npad_i % (SPAD - NSSE)])
    bat_sc = jnp.concatenate([batch.astype(jnp.int32),
                              NG + npad_i % (GPAD - NG)])

    r2 = lambda w: w.reshape(1, -1)

    def LN(p):
        return r2(p['g']), r2(p['b'])

    def GW(p):
        return (p['Wh'], p['Ws'], r2(p['bs']), p['Wv'], p['Wg'], r2(p['bg']))

    H = _tc_init(xs, xv3, *LN(params['W_v']['ln']),
                 *GW(params['W_v']['gvp']), NPAD, 1024, 64, 4)
    es_a, ev_a = _tc_edge_init(eas, ev8, *LN(params['W_e']['ln']),
                               *GW(params['W_e']['gvp']))
    SS = _tc_init(sss, ssv3, *LN(params['W_sse']['ln']),
                  *GW(params['W_sse']['gvp']), SPAD, 640, 64, 4)

    for lp in params['layers']:
        Gs = _sc_gather(H, src, EPAD)
        Gd = _sc_gather(H, dstg, EPAD)
        M = _tc_msg(Gs, Gd, es_a, ev_a, *GW(lp['msg']))
        Msum = _sc_scatter(M, dst_sc, NPAD, EPAD, 6)
        H2 = _tc_node_upd(H, Msum, *LN(lp['ln1']), *GW(lp['ff']),
                          *LN(lp['ln2']))
        Pool = _sc_scatter(H2, n2s_sc, SPAD, NPAD, 6)
        SS = _tc_sse_upd(SS, Pool, *GW(lp['sse_upd']), *LN(lp['ln_sse']))
        Bc = _sc_gather(SS, n2s_g, NPAD)
        H = _tc_node_sse(H2, Bc, *GW(lp['node_sse']), *LN(lp['ln3']))

    ne, neg = _tc_out(H, *LN(params['W_out']['ln']),
                      params['W_out']['gvp']['Wh'],
                      params['W_out']['gvp']['Ws'],
                      r2(params['W_out']['gvp']['bs']))
    Gp = _sc_scatter(neg, bat_sc, GPAD, NPAD, 4)
    ge = _tc_graph_combine(Gp)
    return ne[:N], ge[:NG]


# speedup vs baseline: 23.0420x; 1.0273x over previous
import functools

import jax
import jax.numpy as jnp
from jax import lax
from jax.experimental import pallas as pl
from jax.experimental.pallas import tpu as pltpu
from jax.experimental.pallas import tpu_sc as plsc

N = 50000
E = 800000
NSSE = 5000
NG = 50

NW = 32
NCORE = 2
NSUB = 16
CH = 128

NPAD = 53248
EPAD = 819200
SPAD = 5120
GPAD = 128

F32 = jnp.float32


def _ln_math(s, V, g, b):
    mu = jnp.mean(s, axis=-1, keepdims=True)
    var = jnp.mean((s - mu) ** 2, axis=-1, keepdims=True)
    s = (s - mu) * lax.rsqrt(var + 1e-5) * g + b
    vn2 = V[0] * V[0] + V[1] * V[1] + V[2] * V[2]
    vnorm = jnp.sqrt(jnp.mean(vn2, axis=-1, keepdims=True) + 1e-8)
    V = [v / vnorm for v in V]
    return s, V


def _gvp_math(s, V, Wh, Ws, bs, Wv=None, Wg=None, bg=None, act=None):
    Vh = [jnp.dot(v, Wh, preferred_element_type=F32) for v in V]
    vn = jnp.sqrt(Vh[0] ** 2 + Vh[1] ** 2 + Vh[2] ** 2 + 1e-8)
    so = jnp.dot(jnp.concatenate([s, vn], axis=-1), Ws,
                 preferred_element_type=F32) + bs
    Vout = None
    if Wv is not None:
        Vout = [jnp.dot(vh, Wv, preferred_element_type=F32) for vh in Vh]
        gin = act(so) if act is not None else so
        gate = jax.nn.sigmoid(jnp.dot(gin, Wg, preferred_element_type=F32) + bg)
        Vout = [v * gate for v in Vout]
    if act is not None:
        so = act(so)
    return so, Vout


def _pack128(so, Vo):
    b = so.shape[0]
    return jnp.concatenate(
        [so] + Vo + [jnp.ones((b, 1), F32), jnp.zeros((b, 39), F32)], axis=-1)


def _rspec(blk, f):
    return pl.BlockSpec((blk, f), lambda i: (i, 0))


def _fspec(shape):
    nd = len(shape)
    return pl.BlockSpec(shape, lambda i: (0,) * nd)


_TC_PARAMS = pltpu.CompilerParams(dimension_semantics=("parallel",))


def _tc_init(s_in, v_in, lg, lb, Wh, Ws, bs, Wv, Wg, bg, nrows, blk, si, vi):
    def body(s_ref, v_ref, lg_r, lb_r, wh_r, ws_r, bs_r, wv_r, wg_r, bg_r,
             o_ref):
        s = s_ref[...].T
        vt = v_ref[...].T
        V = [vt[:, j * vi:(j + 1) * vi] for j in range(3)]
        s, V = _ln_math(s, V, lg_r[...], lb_r[...])
        so, Vo = _gvp_math(s, V, wh_r[...], ws_r[...], bs_r[...],
                           wv_r[...], wg_r[...], bg_r[...], act=None)
        o_ref[...] = _pack128(so, Vo)

    ws = [lg, lb, Wh, Ws, bs, Wv, Wg, bg]
    return pl.pallas_call(
        body, grid=(nrows // blk,),
        in_specs=[pl.BlockSpec((si, blk), lambda i: (0, i)),
                  pl.BlockSpec((16, blk), lambda i: (0, i))] +
                 [_fspec(w.shape) for w in ws],
        out_specs=_rspec(blk, 128),
        out_shape=jax.ShapeDtypeStruct((nrows, 128), F32),
        compiler_params=_TC_PARAMS,
    )(s_in, v_in, *ws)


def _tc_edge_init(ea, ev8, lg, lb, Wh, Ws, bs, Wv, Wg, bg, blk=1024):
    def body(a_ref, v_ref, lg_r, lb_r, wh_r, ws_r, bs_r, wv_r, wg_r, bg_r,
             so_ref, vo_ref):
        s = a_ref[...].T
        vt = v_ref[...].T
        V = [vt[:, j:j + 1] for j in range(3)]
        s, V = _ln_math(s, V, lg_r[...], lb_r[...])
        so, Vo = _gvp_math(s, V, wh_r[...], ws_r[...], bs_r[...],
                           wv_r[...], wg_r[...], bg_r[...], act=None)
        so_ref[...] = so
        vo_ref[...] = jnp.concatenate(
            Vo + [jnp.zeros((so.shape[0], 5), F32)], axis=-1)

    ws = [lg, lb, Wh, Ws, bs, Wv, Wg, bg]
    return pl.pallas_call(
        body, grid=(EPAD // blk,),
        in_specs=[pl.BlockSpec((32, blk), lambda i: (0, i)),
                  pl.BlockSpec((8, blk), lambda i: (0, i))] +
                 [_fspec(w.shape) for w in ws],
        out_specs=[_rspec(blk, 32), _rspec(blk, 8)],
        out_shape=[jax.ShapeDtypeStruct((EPAD, 32), F32),
                   jax.ShapeDtypeStruct((EPAD, 8), F32)],
        compiler_params=_TC_PARAMS,
    )(ea, ev8, *ws)


def _tc_msg(gs, gd, es, ev8, Wh, Ws, bs, Wv, Wg, bg, blk=2048):
    def body(gs_ref, gd_ref, es_ref, ev_ref, wh_r, ws_r, bs_r, wv_r, wg_r,
             bg_r, o_ref):
        a = gs_ref[...]
        b = gd_ref[...]
        s = jnp.concatenate([a[:, :64], es_ref[...], b[:, :64]], axis=-1)
        ev = ev_ref[...]
        V = [jnp.concatenate([a[:, 64 + 8 * j:72 + 8 * j], ev[:, j:j + 1],
                              b[:, 64 + 8 * j:72 + 8 * j]], axis=-1)
             for j in range(3)]
        so, Vo = _gvp_math(s, V, wh_r[...], ws_r[...], bs_r[...],
                           wv_r[...], wg_r[...], bg_r[...], act=jax.nn.relu)
        o_ref[...] = _pack128(so, Vo)

    ws = [Wh, Ws, bs, Wv, Wg, bg]
    return pl.pallas_call(
        body, grid=(EPAD // blk,),
        in_specs=[_rspec(blk, 128), _rspec(blk, 128), _rspec(blk, 32),
                  _rspec(blk, 8)] + [_fspec(w.shape) for w in ws],
        out_specs=_rspec(blk, 128),
        out_shape=jax.ShapeDtypeStruct((EPAD, 128), F32),
        compiler_params=_TC_PARAMS,
    )(gs, gd, es, ev8, *ws)


def _mean_from_partials(m_ref):
    m = m_ref[...]
    m = m[0] + m[1]
    cnt = jnp.maximum(m[:, 88:89], 1.0)
    ms = m[:, :64] / cnt
    mV = [m[:, 64 + 8 * j:72 + 8 * j] / cnt for j in range(3)]
    return ms, mV


def _tc_node_upd(H, Msum, l1g, l1b, Wh, Ws, bs, Wv, Wg, bg, l2g, l2b,
                 blk=1024):
    def body(h_ref, m_ref, l1g_r, l1b_r, wh_r, ws_r, bs_r, wv_r, wg_r, bg_r,
             l2g_r, l2b_r, o_ref):
        h = h_ref[...]
        ms, mV = _mean_from_partials(m_ref)
        s = h[:, :64] + ms
        V = [h[:, 64 + 8 * j:72 + 8 * j] + mV[j] for j in range(3)]
        s, V = _ln_math(s, V, l1g_r[...], l1b_r[...])
        ds, dV = _gvp_math(s, V, wh_r[...], ws_r[...], bs_r[...],
                           wv_r[...], wg_r[...], bg_r[...], act=jax.nn.relu)
        s2, V2 = _ln_math(s + ds, [V[j] + dV[j] for j in range(3)],
                          l2g_r[...], l2b_r[...])
        o_ref[...] = _pack128(s2, V2)

    ws = [l1g, l1b, Wh, Ws, bs, Wv, Wg, bg, l2g, l2b]
    return pl.pallas_call(
        body, grid=(NPAD // blk,),
        in_specs=[_rspec(blk, 128),
                  pl.BlockSpec((2, blk, 128), lambda i: (0, i, 0))] +
                 [_fspec(w.shape) for w in ws],
        out_specs=_rspec(blk, 128),
        out_shape=jax.ShapeDtypeStruct((NPAD, 128), F32),
        compiler_params=_TC_PARAMS,
    )(H, Msum, *ws)


def _tc_sse_upd(SS, Pool, Wh, Ws, bs, Wv, Wg, bg, lg, lb, blk=640):
    def body(ss_ref, p_ref, wh_r, ws_r, bs_r, wv_r, wg_r, bg_r, lg_r, lb_r,
             o_ref):
        h = ss_ref[...]
        ps, pV = _mean_from_partials(p_ref)
        s = jnp.concatenate([h[:, :64], ps], axis=-1)
        V = [jnp.concatenate([h[:, 64 + 8 * j:72 + 8 * j], pV[j]], axis=-1)
             for j in range(3)]
        ds, dV = _gvp_math(s, V, wh_r[...], ws_r[...], bs_r[...],
                           wv_r[...], wg_r[...], bg_r[...], act=jax.nn.relu)
        s2, V2 = _ln_math(h[:, :64] + ds,
                          [h[:, 64 + 8 * j:72 + 8 * j] + dV[j]
                           for j in range(3)], lg_r[...], lb_r[...])
        o_ref[...] = _pack128(s2, V2)

    ws = [Wh, Ws, bs, Wv, Wg, bg, lg, lb]
    return pl.pallas_call(
        body, grid=(SPAD // blk,),
        in_specs=[_rspec(blk, 128),
                  pl.BlockSpec((2, blk, 128), lambda i: (0, i, 0))] +
                 [_fspec(w.shape) for w in ws],
        out_specs=_rspec(blk, 128),
        out_shape=jax.ShapeDtypeStruct((SPAD, 128), F32),
        compiler_params=_TC_PARAMS,
    )(SS, Pool, *ws)


def _tc_node_sse(H2, Bc, Wh, Ws, bs, Wv, Wg, bg, lg, lb, blk=1024):
    def body(h_ref, b_ref, wh_r, ws_r, bs_r, wv_r, wg_r, bg_r, lg_r, lb_r,
             o_ref):
        h = h_ref[...]
        c = b_ref[...]
        s = jnp.concatenate([h[:, :64], c[:, :64]], axis=-1)
        V = [jnp.concatenate([h[:, 64 + 8 * j:72 + 8 * j],
                              c[:, 64 + 8 * j:72 + 8 * j]], axis=-1)
             for j in range(3)]
        ds, dV = _gvp_math(s, V, wh_r[...], ws_r[...], bs_r[...],
                           wv_r[...], wg_r[...], bg_r[...], act=jax.nn.relu)
        s2, V2 = _ln_math(h[:, :64] + ds,
                          [h[:, 64 + 8 * j:72 + 8 * j] + dV[j]
                           for j in range(3)], lg_r[...], lb_r[...])
        o_ref[...] = _pack128(s2, V2)

    ws = [Wh, Ws, bs, Wv, Wg, bg, lg, lb]
    return pl.pallas_call(
        body, grid=(NPAD // blk,),
        in_specs=[_rspec(blk, 128), _rspec(blk, 128)] +
                 [_fspec(w.shape) for w in ws],
        out_specs=_rspec(blk, 128),
        out_shape=jax.ShapeDtypeStruct((NPAD, 128), F32),
        compiler_params=_TC_PARAMS,
    )(H2, Bc, *ws)


def _tc_out(H, lg, lb, Wh, Ws, bs, blk=1024):
    def body(h_ref, lg_r, lb_r, wh_r, ws_r, bs_r, o_ref, og_ref):
        h = h_ref[...]
        s = h[:, :64]
        V = [h[:, 64 + 8 * j:72 + 8 * j] for j in range(3)]
        s, V = _ln_math(s, V, lg_r[...], lb_r[...])
        so, _ = _gvp_math(s, V, wh_r[...], ws_r[...], bs_r[...],
                          act=jax.nn.relu)
        o_ref[...] = so
        og_ref[...] = jnp.concatenate(
            [so, jnp.zeros((so.shape[0], 64), F32)], axis=-1)

    ws = [lg, lb, Wh, Ws, bs]
    return pl.pallas_call(
        body, grid=(NPAD // blk,),
        in_specs=[_rspec(blk, 128)] + [_fspec(w.shape) for w in ws],
        out_specs=[_rspec(blk, 64), _rspec(blk, 128)],
        out_shape=[jax.ShapeDtypeStruct((NPAD, 64), F32),
                   jax.ShapeDtypeStruct((NPAD, 128), F32)],
        compiler_params=_TC_PARAMS,
    )(H, *ws)


def _tc_graph_combine(Gp):
    def body(g_ref, o_ref):
        g = g_ref[...]
        o_ref[...] = (g[0] + g[1])[:, :64]

    return pl.pallas_call(
        body, grid=(1,),
        in_specs=[_fspec((2, GPAD, 128))],
        out_specs=_fspec((GPAD, 64)),
        out_shape=jax.ShapeDtypeStruct((GPAD, 64), F32),
    )(Gp)


_SC_MESH = dict(core_axis_name="c", subcore_axis_name="s")


def _sc_gather(table, idx, nrows_out):
    per_w = nrows_out // NW
    nch = per_w // CH
    fdim = table.shape[1]
    Q = 4
    nq, tail = nch // Q, nch % Q

    @functools.partial(
        pl.kernel,
        mesh=plsc.VectorSubcoreMesh(**_SC_MESH),
        out_type=jax.ShapeDtypeStruct((nrows_out, fdim), F32),
        scratch_types=[pltpu.VMEM((per_w,), jnp.int32)] +
                      [pltpu.VMEM((CH, fdim), F32)] * 4 +
                      [pltpu.SemaphoreType.DMA, pltpu.SemaphoreType.DMA],
    )
    def k(t_ref, i_ref, o_ref, idx_all, r0, r1, r2, r3, sem, semo):
        wid = lax.axis_index("s") * NCORE + lax.axis_index("c")
        base = wid * per_w
        pltpu.sync_copy(i_ref.at[pl.ds(base, per_w)], idx_all)
        rows = (r0, r1, r2, r3)

        def chunk_io(j0, nb):
            hs = []
            for b in range(nb):
                hs.append(pltpu.async_copy(
                    t_ref.at[idx_all.at[pl.ds((j0 + b) * CH, CH)]],
                    rows[b], sem))
            for h in hs:
                h.wait()
            hs = []
            for b in range(nb):
                hs.append(pltpu.async_copy(
                    rows[b], o_ref.at[pl.ds(base + (j0 + b) * CH, CH)],
                    semo))
            for h in hs:
                h.wait()

        def body(j2, carry):
            chunk_io(j2 * Q, Q)
            return carry

        lax.fori_loop(0, nq, body, 0)
        if tail:
            chunk_io(nq * Q, tail)

    return k(table, idx)


def _sc_scatter(msgs, idx, table_rows, nsrc, ngroups):
    per_w = nsrc // NW
    nch = per_w // CH
    rpt = table_rows // NSUB
    zeros = jnp.zeros((rpt, 16), F32)
    Q = 4
    nq, tail = nch // Q, nch % Q

    @functools.partial(
        pl.kernel,
        mesh=plsc.VectorSubcoreMesh(**_SC_MESH),
        out_type=jax.ShapeDtypeStruct((NCORE, table_rows, 128), F32),
        scratch_types=[pltpu.VMEM((CH,), jnp.int32)] * 4 +
                      [pltpu.VMEM((CH, 16), F32)] * 4 +
                      [pltpu.VMEM_SHARED((table_rows, 16), F32),
                       pltpu.SemaphoreType.DMA, pltpu.SemaphoreType.DMA],
        compiler_params=pltpu.CompilerParams(use_tc_tiling_on_sc=False),
    )
    def k(m_ref, i_ref, z_ref, o_ref, i0, i1, i2, i3, v0, v1, v2, v3,
          shared, sem, sema):
        cid = lax.axis_index("c")
        sid = lax.axis_index("s")
        wid = sid * NCORE + cid
        ic = (i0, i1, i2, i3)
        mv = (v0, v1, v2, v3)
        for g in range(ngroups):
            pltpu.sync_copy(z_ref, shared.at[pl.ds(sid * rpt, rpt)])
            plsc.subcore_barrier()

            def chunk_adds(j0, nb):
                hs = []
                for b in range(nb):
                    off = wid * per_w + (j0 + b) * CH
                    hs.append(pltpu.async_copy(
                        i_ref.at[pl.ds(off, CH)], ic[b], sem))
                    hs.append(pltpu.async_copy(
                        m_ref.at[pl.ds(off, CH), pl.ds(16 * g, 16)],
                        mv[b], sem))
                for h in hs:
                    h.wait()
                hs = []
                for b in range(nb):
                    hs.append(pltpu.async_copy(
                        mv[b], shared.at[ic[b]], sema, add=True))
                for h in hs:
                    h.wait()

            def body(j2, carry):
                chunk_adds(j2 * Q, Q)
                return carry

            lax.fori_loop(0, nq, body, 0)
            if tail:
                chunk_adds(nq * Q, tail)
            plsc.subcore_barrier()
            pltpu.sync_copy(shared.at[pl.ds(sid * rpt, rpt)],
                            o_ref.at[cid, pl.ds(sid * rpt, rpt),
                                     pl.ds(16 * g, 16)])
            plsc.subcore_barrier()

    return k(msgs, idx, zeros)


def kernel(x, x_vector_attr, edge_attr, edge_vector_attr, sse_attr,
           sse_vector_attr, params, edge_index, node_to_sse, batch):
    xs = jnp.pad(x.T, ((0, 0), (0, NPAD - N)))
    xv3 = jnp.pad(x_vector_attr.transpose(2, 1, 0).reshape(12, N),
                  ((0, 4), (0, NPAD - N)))
    eas = jnp.pad(edge_attr.T, ((0, 0), (0, EPAD - E)))
    ev8 = jnp.pad(edge_vector_attr.transpose(2, 1, 0).reshape(3, E),
                  ((0, 5), (0, EPAD - E)))
    sss = jnp.pad(sse_attr.T, ((0, 0), (0, SPAD - NSSE)))
    ssv3 = jnp.pad(sse_vector_attr.transpose(2, 1, 0).reshape(12, NSSE),
                   ((0, 4), (0, SPAD - NSSE)))

    epad_i = jnp.arange(EPAD - E, dtype=jnp.int32)
    npad_i = jnp.arange(NPAD - N, dtype=jnp.int32)
    src = jnp.concatenate([edge_index[0].astype(jnp.int32), epad_i % N])
    dstg = jnp.concatenate([edge_index[1].astype(jnp.int32), epad_i % N])
    dst_sc = jnp.concatenate([edge_index[1].astype(jnp.int32),
                              N + epad_i % (NPAD - N)])
    n2s = node_to_sse.astype(jnp.int32)
    n2s_g = jnp.concatenate([n2s, npad_i % NSSE])
    n2s_sc = jnp.concatenate([n2s, NSSE + npad_i % (SPAD - NSSE)])
    bat_sc = jnp.concatenate([batch.astype(jnp.int32),
                              NG + npad_i % (GPAD - NG)])

    r2 = lambda w: w.reshape(1, -1)

    def LN(p):
        return r2(p['g']), r2(p['b'])

    def GW(p):
        return (p['Wh'], p['Ws'], r2(p['bs']), p['Wv'], p['Wg'], r2(p['bg']))

    H = _tc_init(xs, xv3, *LN(params['W_v']['ln']),
                 *GW(params['W_v']['gvp']), NPAD, 1024, 64, 4)
    es_a, ev_a = _tc_edge_init(eas, ev8, *LN(params['W_e']['ln']),
                               *GW(params['W_e']['gvp']))
    SS = _tc_init(sss, ssv3, *LN(params['W_sse']['ln']),
                  *GW(params['W_sse']['gvp']), SPAD, 640, 64, 4)

    for lp in params['layers']:
        Gs = _sc_gather(H, src, EPAD)
        Gd = _sc_gather(H, dstg, EPAD)
        M = _tc_msg(Gs, Gd, es_a, ev_a, *GW(lp['msg']))
        Msum = _sc_scatter(M, dst_sc, NPAD, EPAD, 6)
        H2 = _tc_node_upd(H, Msum, *LN(lp['ln1']), *GW(lp['ff']),
                          *LN(lp['ln2']))
        Pool = _sc_scatter(H2, n2s_sc, SPAD, NPAD, 6)
        SS = _tc_sse_upd(SS, Pool, *GW(lp['sse_upd']), *LN(lp['ln_sse']))
        Bc = _sc_gather(SS, n2s_g, NPAD)
        H = _tc_node_sse(H2, Bc, *GW(lp['node_sse']), *LN(lp['ln3']))

    ne, neg = _tc_out(H, *LN(params['W_out']['ln']),
                      params['W_out']['gvp']['Wh'],
                      params['W_out']['gvp']['Ws'],
                      r2(params['W_out']['gvp']['bs']))
    Gp = _sc_scatter(neg, bat_sc, GPAD, NPAD, 4)
    ge = _tc_graph_combine(Gp)
    return ne[:N], ge[:NG]
